# Initial kernel scaffold; baseline (speedup 1.0000x reference)
import jax
import jax.numpy as jnp
from jax.experimental import pallas as pl


def kernel(xyz, features, w1, b1, w2, b2, wcr, bcr):
    raise NotImplementedError("write your pallas kernel here")


# speedup vs baseline: 89.2930x; 89.2930x over previous
import functools

import jax
import jax.numpy as jnp
import numpy as np
from jax import lax
from jax.experimental import pallas as pl
from jax.experimental.pallas import tpu as pltpu
from jax.experimental.pallas import tpu_sc as plsc

B = 4
N = 8192
P = 1024
S1, S2 = 16, 32
CF = 64
CIN = CF + 3
CMID = 32
COUT = 128
EPS = 1e-5
R1SQ = np.float32(0.1 * 0.1)
R2SQ = np.float32(0.2 * 0.2)
D = 80
T1 = B * P * S1
T2 = B * P * S2

NC, NS = 2, 16
NW = NC * NS
PC = (B * P) // NW


def _fps_body(x_ref, y_ref, z_ref, nx_ref, ny_ref, nz_ref, dists_ref):
    x = x_ref[:, :]
    y = y_ref[:, :]
    z = z_ref[:, :]
    iota = lax.broadcasted_iota(jnp.int32, (B, N), 1)
    piota = lax.broadcasted_iota(jnp.int32, (B, P), 1)
    dists_ref[:, :] = jnp.full((B, N), 1e10, jnp.float32)
    cx0 = x[:, 0:1]
    cy0 = y[:, 0:1]
    cz0 = z[:, 0:1]
    nx_ref[:, :] = jnp.broadcast_to(cx0, (B, P))
    ny_ref[:, :] = jnp.broadcast_to(cy0, (B, P))
    nz_ref[:, :] = jnp.broadcast_to(cz0, (B, P))

    def body(i, cur):
        cx, cy, cz = cur
        dx = x - cx
        dy = y - cy
        dz = z - cz
        d = (dx * dx + dy * dy) + dz * dz
        dists = jnp.minimum(dists_ref[:, :], d)
        dists_ref[:, :] = dists
        m = jnp.max(dists, axis=1, keepdims=True)
        idx = jnp.min(jnp.where(dists == m, iota, N), axis=1, keepdims=True)
        hit = iota == idx
        ncx = jnp.sum(jnp.where(hit, x, 0.0), axis=1, keepdims=True)
        ncy = jnp.sum(jnp.where(hit, y, 0.0), axis=1, keepdims=True)
        ncz = jnp.sum(jnp.where(hit, z, 0.0), axis=1, keepdims=True)
        sel = piota == i
        nx_ref[:, :] = jnp.where(sel, ncx, nx_ref[:, :])
        ny_ref[:, :] = jnp.where(sel, ncy, ny_ref[:, :])
        nz_ref[:, :] = jnp.where(sel, ncz, nz_ref[:, :])
        return (ncx, ncy, ncz)

    lax.fori_loop(1, P, body, (cx0, cy0, cz0))


def _fps(xs, ys, zs):
    return pl.pallas_call(
        _fps_body,
        out_shape=[jax.ShapeDtypeStruct((B, P), jnp.float32)] * 3,
        scratch_shapes=[pltpu.VMEM((B, N), jnp.float32)],
    )(xs, ys, zs)


_SC_MESH = plsc.VectorSubcoreMesh(
    core_axis_name="c", subcore_axis_name="s", num_cores=NC, num_subcores=NS
)


@functools.partial(
    pl.kernel,
    out_type=[
        jax.ShapeDtypeStruct((B * P * S1,), jnp.int32),
        jax.ShapeDtypeStruct((B * P * S2,), jnp.int32),
    ],
    mesh=_SC_MESH,
    compiler_params=pltpu.CompilerParams(needs_layout_passes=False),
    scratch_types=[
        pltpu.VMEM((N,), jnp.float32),
        pltpu.VMEM((N,), jnp.float32),
        pltpu.VMEM((N,), jnp.float32),
        pltpu.VMEM((PC,), jnp.float32),
        pltpu.VMEM((PC,), jnp.float32),
        pltpu.VMEM((PC,), jnp.float32),
        pltpu.VMEM((S1 + 16,), jnp.int32),
        pltpu.VMEM((S2 + 16,), jnp.int32),
        pltpu.VMEM((PC * S1,), jnp.int32),
        pltpu.VMEM((PC * S2,), jnp.int32),
    ],
)
def _ballquery_sc(xs_hbm, ys_hbm, zs_hbm, nxs_hbm, nys_hbm, nzs_hbm,
                  out1_hbm, out2_hbm, xv, yv, zv, nxv, nyv, nzv,
                  c1, c2, o1, o2):
    wid = lax.axis_index("s") * NC + lax.axis_index("c")
    tiles_per_b = NW // B
    b = wid // tiles_per_b
    p0 = (wid % tiles_per_b) * PC
    pltpu.sync_copy(xs_hbm.at[b], xv)
    pltpu.sync_copy(ys_hbm.at[b], yv)
    pltpu.sync_copy(zs_hbm.at[b], zv)
    pltpu.sync_copy(nxs_hbm.at[b, pl.ds(p0, PC)], nxv)
    pltpu.sync_copy(nys_hbm.at[b, pl.ds(p0, PC)], nyv)
    pltpu.sync_copy(nzs_hbm.at[b, pl.ds(p0, PC)], nzv)
    lane = lax.iota(jnp.int32, 16)
    gbase = b * N

    def per_centroid(i, _):
        iv = jnp.full((16,), i, jnp.int32)
        cxv = plsc.load_gather(nxv, [iv])
        cyv = plsc.load_gather(nyv, [iv])
        czv = plsc.load_gather(nzv, [iv])

        def cond(state):
            c, cnt1, cnt2 = state
            return (c < N // 16) & ((cnt1 < S1) | (cnt2 < S2))

        def body(state):
            c, cnt1, cnt2 = state
            dx = cxv - xv[pl.ds(c * 16, 16)]
            dy = cyv - yv[pl.ds(c * 16, 16)]
            dz = czv - zv[pl.ds(c * 16, 16)]
            d2 = (dx * dx + dy * dy) + dz * dz
            jv = c * 16 + lane
            m1 = d2 < R1SQ
            m2 = d2 < R2SQ

            @pl.when(cnt1 < S1)
            def _():
                plsc.store_compressed(c1.at[pl.ds(cnt1, 16)], jv, mask=m1)

            @pl.when(cnt2 < S2)
            def _():
                plsc.store_compressed(c2.at[pl.ds(cnt2, 16)], jv, mask=m2)

            n1 = jnp.sum(m1.astype(jnp.int32))
            n2 = jnp.sum(m2.astype(jnp.int32))
            return (c + 1, cnt1 + n1, cnt2 + n2)

        _, cnt1, cnt2 = lax.while_loop(cond, body, (0, 0, 0))

        true16 = lane < 16
        first1 = jnp.full((16,), c1[pl.ds(0, 16)][0], jnp.int32)
        plsc.store_compressed(c1.at[pl.ds(jnp.minimum(cnt1, S1), 16)],
                              first1, mask=true16)
        o1[pl.ds(i * S1, 16)] = c1[pl.ds(0, 16)] + gbase
        first2 = jnp.full((16,), c2[pl.ds(0, 16)][0], jnp.int32)
        plsc.store_compressed(c2.at[pl.ds(jnp.minimum(cnt2, S2), 16)],
                              first2, mask=true16)
        for k in range(S2 // 16):
            o2[pl.ds(i * S2 + k * 16, 16)] = c2[pl.ds(k * 16, 16)] + gbase
        return 0

    lax.fori_loop(0, PC, per_centroid, 0)
    pltpu.sync_copy(o1, out1_hbm.at[pl.ds(wid * PC * S1, PC * S1)])
    pltpu.sync_copy(o2, out2_hbm.at[pl.ds(wid * PC * S2, PC * S2)])


TT = T1 + T2
GCHUNK = 128


@functools.partial(
    pl.kernel,
    out_type=jax.ShapeDtypeStruct((TT, D), jnp.float32),
    mesh=_SC_MESH,
    compiler_params=pltpu.CompilerParams(
        needs_layout_passes=False, use_tc_tiling_on_sc=False),
    scratch_types=[
        pltpu.VMEM((GCHUNK,), jnp.int32),
        pltpu.VMEM((GCHUNK, D), jnp.float32),
        pltpu.SemaphoreType.DMA,
    ],
)
def _gather_sc(table_hbm, idx_hbm, out_hbm, idx_v, rows_v, sem):
    wid = lax.axis_index("s") * NC + lax.axis_index("c")
    per_w = TT // NW
    base = wid * per_w

    def chunk(k, _):
        off = base + k * GCHUNK
        pltpu.sync_copy(idx_hbm.at[pl.ds(off, GCHUNK)], idx_v)
        pltpu.async_copy(table_hbm.at[idx_v], rows_v, sem).wait()
        pltpu.sync_copy(rows_v, out_hbm.at[pl.ds(off, GCHUNK)])
        return 0

    lax.fori_loop(0, per_w // GCHUNK, chunk, 0)


def _bf(v):
    return v.astype(jnp.bfloat16).astype(jnp.float32)


def _h1_from_tile(gat, aux, w1p):
    raw = gat[:, 0:3]
    delta = raw - aux[:, 0:3]
    d0 = delta[:, 0:1]
    d1 = delta[:, 1:2]
    d2c = delta[:, 2:3]
    dist = jnp.sqrt((d0 * d0 + d1 * d1) + d2c * d2c + 1e-12)
    h1 = w1p[15:16, :]
    h1 = h1 + _bf(dist) * _bf(w1p[0:1, :])
    for k in range(3):
        h1 = h1 + _bf(aux[:, 3 + k:4 + k]) * _bf(w1p[1 + k:2 + k, :])
    for k in range(3):
        h1 = h1 + _bf(gat[:, k:k + 1]) * _bf(w1p[4 + k:5 + k, :])
    h1 = h1 + _bf(d0) * _bf(w1p[7:8, :])
    h1 = h1 + _bf(d1) * _bf(w1p[8:9, :])
    h1 = h1 + _bf(d2c) * _bf(w1p[9:10, :])
    return h1, delta


def _d1_body(gat_ref, aux_ref, w1p_ref, s_ref):
    h1, _ = _h1_from_tile(gat_ref[:, :], aux_ref[:, :], w1p_ref[:, :])
    p0 = jnp.sum(h1, axis=0, keepdims=True)
    p1 = jnp.sum(h1 * h1, axis=0, keepdims=True)
    part = jnp.concatenate([p0, p1], axis=0)

    @pl.when(pl.program_id(0) == 0)
    def _():
        s_ref[:, :] = part

    @pl.when(pl.program_id(0) != 0)
    def _():
        s_ref[:, :] = s_ref[:, :] + part


def _d2_body(gat_ref, aux_ref, w1p_ref, st_ref, w2p_ref, b2p_ref,
             ymax_ref, ys_ref, *, rows, s, t_count):
    h1, delta = _h1_from_tile(gat_ref[:, :], aux_ref[:, :], w1p_ref[:, :])
    st = st_ref[:, :]
    mu = st[0:1, :] * (1.0 / t_count)
    var = st[1:2, :] * (1.0 / t_count) - mu * mu
    rs = 1.0 / jnp.sqrt(var + EPS)
    h1n = jnp.maximum((h1 - mu) * rs, 0.0)
    h2 = jnp.dot(_bf(h1n), _bf(w2p_ref[:, :]),
                 preferred_element_type=jnp.float32,
                 precision=lax.Precision.HIGHEST)
    h2 = h2 + b2p_ref[:, :]
    x = jnp.concatenate(
        [delta, gat_ref[:, 16:80], jnp.zeros((rows, 128 - CIN), jnp.float32)],
        axis=1)
    y = h2 * x
    p0 = jnp.sum(y, axis=0, keepdims=True)
    p1 = jnp.sum(y * y, axis=0, keepdims=True)
    part = jnp.concatenate([p0, p1], axis=0)

    @pl.when(pl.program_id(0) == 0)
    def _():
        ys_ref[:, :] = part

    @pl.when(pl.program_id(0) != 0)
    def _():
        ys_ref[:, :] = ys_ref[:, :] + part

    for k in range(rows // s):
        ymax_ref[k:k + 1, :] = jnp.max(y[k * s:(k + 1) * s, :], axis=0,
                                       keepdims=True)


def _e_body(ymax_ref, ys_ref, wcrp_ref, bcr_ref, out_ref, *, t_count):
    st = ys_ref[:, :]
    mu = st[0:1, :] * (1.0 / t_count)
    var = st[1:2, :] * (1.0 / t_count) - mu * mu
    rs = 1.0 / jnp.sqrt(var + EPS)
    yn = jnp.maximum((ymax_ref[:, :] - mu) * rs, 0.0)
    z = jnp.dot(_bf(yn), _bf(wcrp_ref[:, :]),
                preferred_element_type=jnp.float32,
                precision=lax.Precision.HIGHEST)
    z = z + bcr_ref[:, :]
    muz = jnp.sum(z, axis=0, keepdims=True) * (1.0 / (B * P))
    varz = jnp.sum(z * z, axis=0, keepdims=True) * (1.0 / (B * P)) - muz * muz
    out_ref[:, :] = jnp.maximum((z - muz) / jnp.sqrt(varz + EPS), 0.0)


def _rsconv_scale(gat, aux, w1p, w2p, b2p, wcrp, bcrp, s):
    t_rows = gat.shape[0]
    rows = 512
    grid = (t_rows // rows,)
    stats = pl.pallas_call(
        _d1_body,
        grid=grid,
        in_specs=[
            pl.BlockSpec((rows, D), lambda i: (i, 0)),
            pl.BlockSpec((rows, 8), lambda i: (i, 0)),
            pl.BlockSpec((16, CMID), lambda i: (0, 0)),
        ],
        out_specs=pl.BlockSpec((2, CMID), lambda i: (0, 0)),
        out_shape=jax.ShapeDtypeStruct((2, CMID), jnp.float32),
    )(gat, aux, w1p)
    ymax, ys = pl.pallas_call(
        functools.partial(_d2_body, rows=rows, s=s, t_count=float(t_rows)),
        grid=grid,
        in_specs=[
            pl.BlockSpec((rows, D), lambda i: (i, 0)),
            pl.BlockSpec((rows, 8), lambda i: (i, 0)),
            pl.BlockSpec((16, CMID), lambda i: (0, 0)),
            pl.BlockSpec((2, CMID), lambda i: (0, 0)),
            pl.BlockSpec((CMID, 128), lambda i: (0, 0)),
            pl.BlockSpec((1, 128), lambda i: (0, 0)),
        ],
        out_specs=[
            pl.BlockSpec((rows // s, 128), lambda i: (i, 0)),
            pl.BlockSpec((2, 128), lambda i: (0, 0)),
        ],
        out_shape=[
            jax.ShapeDtypeStruct((B * P, 128), jnp.float32),
            jax.ShapeDtypeStruct((2, 128), jnp.float32),
        ],
    )(gat, aux, w1p, stats, w2p, b2p)
    out = pl.pallas_call(
        functools.partial(_e_body, t_count=float(t_rows)),
        in_specs=[
            pl.BlockSpec((B * P, 128), lambda: (0, 0)),
            pl.BlockSpec((2, 128), lambda: (0, 0)),
            pl.BlockSpec((128, 128), lambda: (0, 0)),
            pl.BlockSpec((1, 128), lambda: (0, 0)),
        ],
        out_specs=pl.BlockSpec((B * P, 128), lambda: (0, 0)),
        out_shape=jax.ShapeDtypeStruct((B * P, 128), jnp.float32),
    )(ymax, ys, wcrp, bcrp)
    return out


def kernel(xyz, features, w1, b1, w2, b2, wcr, bcr):
    xs = xyz[:, :, 0]
    ys = xyz[:, :, 1]
    zs = xyz[:, :, 2]
    nx, ny, nz = _fps(xs, ys, zs)
    new_xyz = jnp.stack([nx, ny, nz], axis=-1)

    idx1, idx2 = _ballquery_sc(xs, ys, zs, nx, ny, nz)

    table = jnp.concatenate(
        [xyz, jnp.zeros((B, N, 13), jnp.float32),
         jnp.transpose(features, (0, 2, 1))], axis=2).reshape(B * N, D)
    gat = _gather_sc(table, jnp.concatenate([idx1, idx2]))
    gat1 = gat[:T1]
    gat2 = gat[T1:]

    newx_flat = new_xyz.reshape(B * P, 3)

    def make_aux(g, s):
        nrep = jnp.repeat(newx_flat, s, axis=0)
        xi = jnp.repeat(g.reshape(B * P, s, D)[:, 0, 0:3], s, axis=0)
        return jnp.concatenate(
            [nrep, xi, jnp.zeros((B * P * s, 2), jnp.float32)], axis=1)

    w1p = jnp.zeros((16, CMID), jnp.float32).at[0:10].set(w1.T).at[15].set(b1)
    w2p = jnp.zeros((CMID, 128), jnp.float32).at[:, 0:CIN].set(w2.T)
    b2p = jnp.zeros((1, 128), jnp.float32).at[:, 0:CIN].set(b2)
    wcrp = jnp.zeros((128, 128), jnp.float32).at[0:CIN].set(wcr.T)
    bcrp = bcr.reshape(1, 128)

    o1 = _rsconv_scale(gat1, make_aux(gat1, S1), w1p, w2p, b2p, wcrp, bcrp, S1)
    o2 = _rsconv_scale(gat2, make_aux(gat2, S2), w1p, w2p, b2p, wcrp, bcrp, S2)
    out = jnp.concatenate([
        o1.reshape(B, P, COUT).transpose(0, 2, 1),
        o2.reshape(B, P, COUT).transpose(0, 2, 1),
    ], axis=1)
    return (new_xyz, out)


# speedup vs baseline: 105.3497x; 1.1798x over previous
import functools

import jax
import jax.numpy as jnp
import numpy as np
from jax import lax
from jax.experimental import pallas as pl
from jax.experimental.pallas import tpu as pltpu
from jax.experimental.pallas import tpu_sc as plsc

B = 4
N = 8192
P = 1024
S1, S2 = 16, 32
CF = 64
CIN = CF + 3
CMID = 32
COUT = 128
EPS = 1e-5
R1SQ = np.float32(0.1 * 0.1)
R2SQ = np.float32(0.2 * 0.2)
D = 80
T1 = B * P * S1
T2 = B * P * S2

NC, NS = 2, 16
NW = NC * NS
PC = (B * P) // NW


def _fps_body(x_ref, y_ref, z_ref, nx_ref, ny_ref, nz_ref, dists_ref):
    x = x_ref[:, :]
    y = y_ref[:, :]
    z = z_ref[:, :]
    iota = lax.broadcasted_iota(jnp.int32, (B, N), 1)
    piota = lax.broadcasted_iota(jnp.int32, (B, P), 1)
    dists_ref[:, :] = jnp.full((B, N), 1e10, jnp.float32)
    cx0 = x[:, 0:1]
    cy0 = y[:, 0:1]
    cz0 = z[:, 0:1]
    nx_ref[:, :] = jnp.broadcast_to(cx0, (B, P))
    ny_ref[:, :] = jnp.broadcast_to(cy0, (B, P))
    nz_ref[:, :] = jnp.broadcast_to(cz0, (B, P))

    def body(i, cur):
        cx, cy, cz = cur
        dx = x - cx
        dy = y - cy
        dz = z - cz
        d = (dx * dx + dy * dy) + dz * dz
        dists = jnp.minimum(dists_ref[:, :], d)
        dists_ref[:, :] = dists
        m = jnp.max(dists, axis=1, keepdims=True)
        idx = jnp.min(jnp.where(dists == m, iota, N), axis=1, keepdims=True)
        hit = iota == idx
        ncx = jnp.sum(jnp.where(hit, x, 0.0), axis=1, keepdims=True)
        ncy = jnp.sum(jnp.where(hit, y, 0.0), axis=1, keepdims=True)
        ncz = jnp.sum(jnp.where(hit, z, 0.0), axis=1, keepdims=True)
        sel = piota == i
        nx_ref[:, :] = jnp.where(sel, ncx, nx_ref[:, :])
        ny_ref[:, :] = jnp.where(sel, ncy, ny_ref[:, :])
        nz_ref[:, :] = jnp.where(sel, ncz, nz_ref[:, :])
        return (ncx, ncy, ncz)

    lax.fori_loop(1, P, body, (cx0, cy0, cz0))


def _fps(xs, ys, zs):
    return pl.pallas_call(
        _fps_body,
        out_shape=[jax.ShapeDtypeStruct((B, P), jnp.float32)] * 3,
        scratch_shapes=[pltpu.VMEM((B, N), jnp.float32)],
    )(xs, ys, zs)


_SC_MESH = plsc.VectorSubcoreMesh(
    core_axis_name="c", subcore_axis_name="s", num_cores=NC, num_subcores=NS
)


@functools.partial(
    pl.kernel,
    out_type=[
        jax.ShapeDtypeStruct((B * P * S1,), jnp.int32),
        jax.ShapeDtypeStruct((B * P * S2,), jnp.int32),
    ],
    mesh=_SC_MESH,
    compiler_params=pltpu.CompilerParams(needs_layout_passes=False),
    scratch_types=[
        pltpu.VMEM((N,), jnp.float32),
        pltpu.VMEM((N,), jnp.float32),
        pltpu.VMEM((N,), jnp.float32),
        pltpu.VMEM((PC,), jnp.float32),
        pltpu.VMEM((PC,), jnp.float32),
        pltpu.VMEM((PC,), jnp.float32),
        pltpu.VMEM((N + 32,), jnp.int32),
        pltpu.VMEM((N + 32,), jnp.int32),
        pltpu.VMEM((PC * S1,), jnp.int32),
        pltpu.VMEM((PC * S2,), jnp.int32),
    ],
)
def _ballquery_sc(xs_hbm, ys_hbm, zs_hbm, nxs_hbm, nys_hbm, nzs_hbm,
                  out1_hbm, out2_hbm, xv, yv, zv, nxv, nyv, nzv,
                  c1, c2, o1, o2):
    wid = lax.axis_index("s") * NC + lax.axis_index("c")
    tiles_per_b = NW // B
    b = wid // tiles_per_b
    p0 = (wid % tiles_per_b) * PC
    pltpu.sync_copy(xs_hbm.at[b], xv)
    pltpu.sync_copy(ys_hbm.at[b], yv)
    pltpu.sync_copy(zs_hbm.at[b], zv)
    pltpu.sync_copy(nxs_hbm.at[b, pl.ds(p0, PC)], nxv)
    pltpu.sync_copy(nys_hbm.at[b, pl.ds(p0, PC)], nyv)
    pltpu.sync_copy(nzs_hbm.at[b, pl.ds(p0, PC)], nzv)
    lane = lax.iota(jnp.int32, 16)
    gbase = b * N

    def per_centroid(i, _):
        iv = jnp.full((16,), i, jnp.int32)
        cxv = plsc.load_gather(nxv, [iv])
        cyv = plsc.load_gather(nyv, [iv])
        czv = plsc.load_gather(nzv, [iv])

        def cond(state):
            c, cnt1, cnt2 = state
            return (c < N // 32) & ((cnt1 < S1) | (cnt2 < S2))

        def body(state):
            c, cnt1, cnt2 = state
            for u in range(2):
                off = c * 32 + u * 16
                dx = cxv - xv[pl.ds(off, 16)]
                dy = cyv - yv[pl.ds(off, 16)]
                dz = czv - zv[pl.ds(off, 16)]
                d2 = (dx * dx + dy * dy) + dz * dz
                jv = off + lane
                m1 = d2 < R1SQ
                m2 = d2 < R2SQ
                plsc.store_compressed(c1.at[pl.ds(cnt1, 16)], jv, mask=m1)
                plsc.store_compressed(c2.at[pl.ds(cnt2, 16)], jv, mask=m2)
                cnt1 = cnt1 + plsc.all_reduce_population_count(m1)[0]
                cnt2 = cnt2 + plsc.all_reduce_population_count(m2)[0]
            return (c + 1, cnt1, cnt2)

        _, cnt1, cnt2 = lax.while_loop(cond, body, (0, 0, 0))

        true16 = lane < 16
        first1 = jnp.full((16,), c1[pl.ds(0, 16)][0], jnp.int32)
        plsc.store_compressed(c1.at[pl.ds(jnp.minimum(cnt1, S1), 16)],
                              first1, mask=true16)
        o1[pl.ds(i * S1, 16)] = c1[pl.ds(0, 16)] + gbase
        first2 = jnp.full((16,), c2[pl.ds(0, 16)][0], jnp.int32)
        plsc.store_compressed(c2.at[pl.ds(jnp.minimum(cnt2, S2), 16)],
                              first2, mask=true16)
        for k in range(S2 // 16):
            o2[pl.ds(i * S2 + k * 16, 16)] = c2[pl.ds(k * 16, 16)] + gbase
        return 0

    lax.fori_loop(0, PC, per_centroid, 0)
    pltpu.sync_copy(o1, out1_hbm.at[pl.ds(wid * PC * S1, PC * S1)])
    pltpu.sync_copy(o2, out2_hbm.at[pl.ds(wid * PC * S2, PC * S2)])


TT = T1 + T2
GCHUNK = 128


@functools.partial(
    pl.kernel,
    out_type=jax.ShapeDtypeStruct((TT, D), jnp.float32),
    mesh=_SC_MESH,
    compiler_params=pltpu.CompilerParams(
        needs_layout_passes=False, use_tc_tiling_on_sc=False),
    scratch_types=[
        pltpu.VMEM((GCHUNK,), jnp.int32),
        pltpu.VMEM((GCHUNK, D), jnp.float32),
        pltpu.SemaphoreType.DMA,
    ],
)
def _gather_sc(table_hbm, idx_hbm, out_hbm, idx_v, rows_v, sem):
    wid = lax.axis_index("s") * NC + lax.axis_index("c")
    per_w = TT // NW
    base = wid * per_w

    def chunk(k, _):
        off = base + k * GCHUNK
        pltpu.sync_copy(idx_hbm.at[pl.ds(off, GCHUNK)], idx_v)
        pltpu.async_copy(table_hbm.at[idx_v], rows_v, sem).wait()
        pltpu.sync_copy(rows_v, out_hbm.at[pl.ds(off, GCHUNK)])
        return 0

    lax.fori_loop(0, per_w // GCHUNK, chunk, 0)


def _bf(v):
    return v.astype(jnp.bfloat16).astype(jnp.float32)


def _h1_from_tile(gat, aux, w1p):
    raw = gat[:, 0:3]
    delta = raw - aux[:, 0:3]
    d0 = delta[:, 0:1]
    d1 = delta[:, 1:2]
    d2c = delta[:, 2:3]
    dist = jnp.sqrt((d0 * d0 + d1 * d1) + d2c * d2c + 1e-12)
    h1 = w1p[15:16, :]
    h1 = h1 + _bf(dist) * _bf(w1p[0:1, :])
    for k in range(3):
        h1 = h1 + _bf(aux[:, 3 + k:4 + k]) * _bf(w1p[1 + k:2 + k, :])
    for k in range(3):
        h1 = h1 + _bf(gat[:, k:k + 1]) * _bf(w1p[4 + k:5 + k, :])
    h1 = h1 + _bf(d0) * _bf(w1p[7:8, :])
    h1 = h1 + _bf(d1) * _bf(w1p[8:9, :])
    h1 = h1 + _bf(d2c) * _bf(w1p[9:10, :])
    return h1, delta


def _d1_body(gat_ref, aux_ref, w1p_ref, s_ref):
    h1, _ = _h1_from_tile(gat_ref[:, :], aux_ref[:, :], w1p_ref[:, :])
    p0 = jnp.sum(h1, axis=0, keepdims=True)
    p1 = jnp.sum(h1 * h1, axis=0, keepdims=True)
    part = jnp.concatenate([p0, p1], axis=0)

    @pl.when(pl.program_id(0) == 0)
    def _():
        s_ref[:, :] = part

    @pl.when(pl.program_id(0) != 0)
    def _():
        s_ref[:, :] = s_ref[:, :] + part


def _d2_body(gat_ref, aux_ref, w1p_ref, st_ref, w2p_ref, b2p_ref,
             ymax_ref, ys_ref, *, rows, s, t_count):
    h1, delta = _h1_from_tile(gat_ref[:, :], aux_ref[:, :], w1p_ref[:, :])
    st = st_ref[:, :]
    mu = st[0:1, :] * (1.0 / t_count)
    var = st[1:2, :] * (1.0 / t_count) - mu * mu
    rs = 1.0 / jnp.sqrt(var + EPS)
    h1n = jnp.maximum((h1 - mu) * rs, 0.0)
    h2 = jnp.dot(_bf(h1n), _bf(w2p_ref[:, :]),
                 preferred_element_type=jnp.float32,
                 precision=lax.Precision.HIGHEST)
    h2 = h2 + b2p_ref[:, :]
    x = jnp.concatenate(
        [delta, gat_ref[:, 16:80], jnp.zeros((rows, 128 - CIN), jnp.float32)],
        axis=1)
    y = h2 * x
    p0 = jnp.sum(y, axis=0, keepdims=True)
    p1 = jnp.sum(y * y, axis=0, keepdims=True)
    part = jnp.concatenate([p0, p1], axis=0)

    @pl.when(pl.program_id(0) == 0)
    def _():
        ys_ref[:, :] = part

    @pl.when(pl.program_id(0) != 0)
    def _():
        ys_ref[:, :] = ys_ref[:, :] + part

    for k in range(rows // s):
        ymax_ref[k:k + 1, :] = jnp.max(y[k * s:(k + 1) * s, :], axis=0,
                                       keepdims=True)


def _e_body(ymax_ref, ys_ref, wcrp_ref, bcr_ref, out_ref, *, t_count):
    st = ys_ref[:, :]
    mu = st[0:1, :] * (1.0 / t_count)
    var = st[1:2, :] * (1.0 / t_count) - mu * mu
    rs = 1.0 / jnp.sqrt(var + EPS)
    yn = jnp.maximum((ymax_ref[:, :] - mu) * rs, 0.0)
    z = jnp.dot(_bf(yn), _bf(wcrp_ref[:, :]),
                preferred_element_type=jnp.float32,
                precision=lax.Precision.HIGHEST)
    z = z + bcr_ref[:, :]
    muz = jnp.sum(z, axis=0, keepdims=True) * (1.0 / (B * P))
    varz = jnp.sum(z * z, axis=0, keepdims=True) * (1.0 / (B * P)) - muz * muz
    out_ref[:, :] = jnp.maximum((z - muz) / jnp.sqrt(varz + EPS), 0.0)


def _rsconv_scale(gat, aux, w1p, w2p, b2p, wcrp, bcrp, s):
    t_rows = gat.shape[0]
    rows = 512
    grid = (t_rows // rows,)
    stats = pl.pallas_call(
        _d1_body,
        grid=grid,
        in_specs=[
            pl.BlockSpec((rows, D), lambda i: (i, 0)),
            pl.BlockSpec((rows, 8), lambda i: (i, 0)),
            pl.BlockSpec((16, CMID), lambda i: (0, 0)),
        ],
        out_specs=pl.BlockSpec((2, CMID), lambda i: (0, 0)),
        out_shape=jax.ShapeDtypeStruct((2, CMID), jnp.float32),
    )(gat, aux, w1p)
    ymax, ys = pl.pallas_call(
        functools.partial(_d2_body, rows=rows, s=s, t_count=float(t_rows)),
        grid=grid,
        in_specs=[
            pl.BlockSpec((rows, D), lambda i: (i, 0)),
            pl.BlockSpec((rows, 8), lambda i: (i, 0)),
            pl.BlockSpec((16, CMID), lambda i: (0, 0)),
            pl.BlockSpec((2, CMID), lambda i: (0, 0)),
            pl.BlockSpec((CMID, 128), lambda i: (0, 0)),
            pl.BlockSpec((1, 128), lambda i: (0, 0)),
        ],
        out_specs=[
            pl.BlockSpec((rows // s, 128), lambda i: (i, 0)),
            pl.BlockSpec((2, 128), lambda i: (0, 0)),
        ],
        out_shape=[
            jax.ShapeDtypeStruct((B * P, 128), jnp.float32),
            jax.ShapeDtypeStruct((2, 128), jnp.float32),
        ],
    )(gat, aux, w1p, stats, w2p, b2p)
    out = pl.pallas_call(
        functools.partial(_e_body, t_count=float(t_rows)),
        in_specs=[
            pl.BlockSpec((B * P, 128), lambda: (0, 0)),
            pl.BlockSpec((2, 128), lambda: (0, 0)),
            pl.BlockSpec((128, 128), lambda: (0, 0)),
            pl.BlockSpec((1, 128), lambda: (0, 0)),
        ],
        out_specs=pl.BlockSpec((B * P, 128), lambda: (0, 0)),
        out_shape=jax.ShapeDtypeStruct((B * P, 128), jnp.float32),
    )(ymax, ys, wcrp, bcrp)
    return out


def kernel(xyz, features, w1, b1, w2, b2, wcr, bcr):
    xs = xyz[:, :, 0]
    ys = xyz[:, :, 1]
    zs = xyz[:, :, 2]
    nx, ny, nz = _fps(xs, ys, zs)
    new_xyz = jnp.stack([nx, ny, nz], axis=-1)

    idx1, idx2 = _ballquery_sc(xs, ys, zs, nx, ny, nz)

    table = jnp.concatenate(
        [xyz, jnp.zeros((B, N, 13), jnp.float32),
         jnp.transpose(features, (0, 2, 1))], axis=2).reshape(B * N, D)
    gat = _gather_sc(table, jnp.concatenate([idx1, idx2]))
    gat1 = gat[:T1]
    gat2 = gat[T1:]

    newx_flat = new_xyz.reshape(B * P, 3)

    def make_aux(g, s):
        nrep = jnp.repeat(newx_flat, s, axis=0)
        xi = jnp.repeat(g.reshape(B * P, s, D)[:, 0, 0:3], s, axis=0)
        return jnp.concatenate(
            [nrep, xi, jnp.zeros((B * P * s, 2), jnp.float32)], axis=1)

    w1p = jnp.zeros((16, CMID), jnp.float32).at[0:10].set(w1.T).at[15].set(b1)
    w2p = jnp.zeros((CMID, 128), jnp.float32).at[:, 0:CIN].set(w2.T)
    b2p = jnp.zeros((1, 128), jnp.float32).at[:, 0:CIN].set(b2)
    wcrp = jnp.zeros((128, 128), jnp.float32).at[0:CIN].set(wcr.T)
    bcrp = bcr.reshape(1, 128)

    o1 = _rsconv_scale(gat1, make_aux(gat1, S1), w1p, w2p, b2p, wcrp, bcrp, S1)
    o2 = _rsconv_scale(gat2, make_aux(gat2, S2), w1p, w2p, b2p, wcrp, bcrp, S2)
    out = jnp.concatenate([
        o1.reshape(B, P, COUT).transpose(0, 2, 1),
        o2.reshape(B, P, COUT).transpose(0, 2, 1),
    ], axis=1)
    return (new_xyz, out)


# speedup vs baseline: 108.8573x; 1.0333x over previous
import functools

import jax
import jax.numpy as jnp
import numpy as np
from jax import lax
from jax.experimental import pallas as pl
from jax.experimental.pallas import tpu as pltpu
from jax.experimental.pallas import tpu_sc as plsc

B = 4
N = 8192
P = 1024
S1, S2 = 16, 32
CF = 64
CIN = CF + 3
CMID = 32
COUT = 128
EPS = 1e-5
R1SQ = np.float32(0.1 * 0.1)
R2SQ = np.float32(0.2 * 0.2)
D = 80
T1 = B * P * S1
T2 = B * P * S2

NC, NS = 2, 16
NW = NC * NS
PC = (B * P) // NW


def _fps_body(x_ref, y_ref, z_ref, nx_ref, ny_ref, nz_ref, dists_ref):
    x = x_ref[:, :]
    y = y_ref[:, :]
    z = z_ref[:, :]
    iota = lax.broadcasted_iota(jnp.int32, (B, N), 1)
    piota = lax.broadcasted_iota(jnp.int32, (B, P), 1)
    dists_ref[:, :] = jnp.full((B, N), 1e10, jnp.float32)
    cx0 = x[:, 0:1]
    cy0 = y[:, 0:1]
    cz0 = z[:, 0:1]
    nx_ref[:, :] = jnp.broadcast_to(cx0, (B, P))
    ny_ref[:, :] = jnp.broadcast_to(cy0, (B, P))
    nz_ref[:, :] = jnp.broadcast_to(cz0, (B, P))

    def body(i, cur):
        cx, cy, cz = cur
        dx = x - cx
        dy = y - cy
        dz = z - cz
        d = (dx * dx + dy * dy) + dz * dz
        dists = jnp.minimum(dists_ref[:, :], d)
        dists_ref[:, :] = dists
        m = jnp.max(dists, axis=1, keepdims=True)
        idx = jnp.min(jnp.where(dists == m, iota, N), axis=1, keepdims=True)
        hit = iota == idx
        ncx = jnp.sum(jnp.where(hit, x, 0.0), axis=1, keepdims=True)
        ncy = jnp.sum(jnp.where(hit, y, 0.0), axis=1, keepdims=True)
        ncz = jnp.sum(jnp.where(hit, z, 0.0), axis=1, keepdims=True)
        sel = piota == i
        nx_ref[:, :] = jnp.where(sel, ncx, nx_ref[:, :])
        ny_ref[:, :] = jnp.where(sel, ncy, ny_ref[:, :])
        nz_ref[:, :] = jnp.where(sel, ncz, nz_ref[:, :])
        return (ncx, ncy, ncz)

    lax.fori_loop(1, P, body, (cx0, cy0, cz0))


def _fps(xs, ys, zs):
    return pl.pallas_call(
        _fps_body,
        out_shape=[jax.ShapeDtypeStruct((B, P), jnp.float32)] * 3,
        scratch_shapes=[pltpu.VMEM((B, N), jnp.float32)],
    )(xs, ys, zs)


_SC_MESH = plsc.VectorSubcoreMesh(
    core_axis_name="c", subcore_axis_name="s", num_cores=NC, num_subcores=NS
)


@functools.partial(
    pl.kernel,
    out_type=[
        jax.ShapeDtypeStruct((B * P * S1,), jnp.int32),
        jax.ShapeDtypeStruct((B * P * S2,), jnp.int32),
    ],
    mesh=_SC_MESH,
    compiler_params=pltpu.CompilerParams(needs_layout_passes=False),
    scratch_types=[
        pltpu.VMEM((N,), jnp.float32),
        pltpu.VMEM((N,), jnp.float32),
        pltpu.VMEM((N,), jnp.float32),
        pltpu.VMEM((PC,), jnp.float32),
        pltpu.VMEM((PC,), jnp.float32),
        pltpu.VMEM((PC,), jnp.float32),
        pltpu.VMEM((N + 32,), jnp.int32),
        pltpu.VMEM((N + 32,), jnp.int32),
        pltpu.VMEM((PC * S1,), jnp.int32),
        pltpu.VMEM((PC * S2,), jnp.int32),
    ],
)
def _ballquery_sc(xs_hbm, ys_hbm, zs_hbm, nxs_hbm, nys_hbm, nzs_hbm,
                  out1_hbm, out2_hbm, xv, yv, zv, nxv, nyv, nzv,
                  c1, c2, o1, o2):
    wid = lax.axis_index("s") * NC + lax.axis_index("c")
    tiles_per_b = NW // B
    b = wid // tiles_per_b
    p0 = (wid % tiles_per_b) * PC
    pltpu.sync_copy(xs_hbm.at[b], xv)
    pltpu.sync_copy(ys_hbm.at[b], yv)
    pltpu.sync_copy(zs_hbm.at[b], zv)
    pltpu.sync_copy(nxs_hbm.at[b, pl.ds(p0, PC)], nxv)
    pltpu.sync_copy(nys_hbm.at[b, pl.ds(p0, PC)], nyv)
    pltpu.sync_copy(nzs_hbm.at[b, pl.ds(p0, PC)], nzv)
    lane = lax.iota(jnp.int32, 16)
    gbase = b * N

    def per_centroid(i, _):
        iv = jnp.full((16,), i, jnp.int32)
        cxv = plsc.load_gather(nxv, [iv])
        cyv = plsc.load_gather(nyv, [iv])
        czv = plsc.load_gather(nzv, [iv])

        def body_a(state):
            c, cnt1, cnt2 = state
            for u in range(2):
                off = c * 32 + u * 16
                dx = cxv - xv[pl.ds(off, 16)]
                dy = cyv - yv[pl.ds(off, 16)]
                dz = czv - zv[pl.ds(off, 16)]
                d2 = (dx * dx + dy * dy) + dz * dz
                jv = off + lane
                m1 = d2 < R1SQ
                m2 = d2 < R2SQ
                plsc.store_compressed(c1.at[pl.ds(cnt1, 16)], jv, mask=m1)
                plsc.store_compressed(c2.at[pl.ds(cnt2, 16)], jv, mask=m2)
                cnt1 = cnt1 + plsc.all_reduce_population_count(m1)[0]
                cnt2 = cnt2 + plsc.all_reduce_population_count(m2)[0]
            return (c + 1, cnt1, cnt2)

        def cond_a2(state):
            c, cnt1, cnt2 = state
            return (c < N // 32) & (cnt2 < S2)

        c_a, cnt1, cnt2 = lax.while_loop(cond_a2, body_a, (0, 0, 0))

        def cond_b(state):
            c, cnt1 = state
            return (c < N // 32) & (cnt1 < S1)

        def body_b(state):
            c, cnt1 = state
            for u in range(2):
                off = c * 32 + u * 16
                dx = cxv - xv[pl.ds(off, 16)]
                dy = cyv - yv[pl.ds(off, 16)]
                dz = czv - zv[pl.ds(off, 16)]
                d2 = (dx * dx + dy * dy) + dz * dz
                m1 = d2 < R1SQ
                plsc.store_compressed(c1.at[pl.ds(cnt1, 16)], off + lane,
                                      mask=m1)
                cnt1 = cnt1 + plsc.all_reduce_population_count(m1)[0]
            return (c + 1, cnt1)

        _, cnt1 = lax.while_loop(cond_b, body_b, (c_a, cnt1))

        true16 = lane < 16
        first1 = jnp.full((16,), c1[pl.ds(0, 16)][0], jnp.int32)
        plsc.store_compressed(c1.at[pl.ds(jnp.minimum(cnt1, S1), 16)],
                              first1, mask=true16)
        o1[pl.ds(i * S1, 16)] = c1[pl.ds(0, 16)] + gbase
        first2 = jnp.full((16,), c2[pl.ds(0, 16)][0], jnp.int32)
        plsc.store_compressed(c2.at[pl.ds(jnp.minimum(cnt2, S2), 16)],
                              first2, mask=true16)
        for k in range(S2 // 16):
            o2[pl.ds(i * S2 + k * 16, 16)] = c2[pl.ds(k * 16, 16)] + gbase
        return 0

    lax.fori_loop(0, PC, per_centroid, 0)
    pltpu.sync_copy(o1, out1_hbm.at[pl.ds(wid * PC * S1, PC * S1)])
    pltpu.sync_copy(o2, out2_hbm.at[pl.ds(wid * PC * S2, PC * S2)])


TT = T1 + T2
GCHUNK = 128


@functools.partial(
    pl.kernel,
    out_type=jax.ShapeDtypeStruct((TT, D), jnp.float32),
    mesh=_SC_MESH,
    compiler_params=pltpu.CompilerParams(
        needs_layout_passes=False, use_tc_tiling_on_sc=False),
    scratch_types=[
        pltpu.VMEM((GCHUNK,), jnp.int32),
        pltpu.VMEM((GCHUNK, D), jnp.float32),
        pltpu.SemaphoreType.DMA,
    ],
)
def _gather_sc(table_hbm, idx_hbm, out_hbm, idx_v, rows_v, sem):
    wid = lax.axis_index("s") * NC + lax.axis_index("c")
    per_w = TT // NW
    base = wid * per_w

    def chunk(k, _):
        off = base + k * GCHUNK
        pltpu.sync_copy(idx_hbm.at[pl.ds(off, GCHUNK)], idx_v)
        pltpu.async_copy(table_hbm.at[idx_v], rows_v, sem).wait()
        pltpu.sync_copy(rows_v, out_hbm.at[pl.ds(off, GCHUNK)])
        return 0

    lax.fori_loop(0, per_w // GCHUNK, chunk, 0)


def _bf(v):
    return v.astype(jnp.bfloat16).astype(jnp.float32)


def _h1_from_tile(gat, aux, w1p):
    raw = gat[:, 0:3]
    delta = raw - aux[:, 0:3]
    d0 = delta[:, 0:1]
    d1 = delta[:, 1:2]
    d2c = delta[:, 2:3]
    dist = jnp.sqrt((d0 * d0 + d1 * d1) + d2c * d2c + 1e-12)
    h1 = w1p[15:16, :]
    h1 = h1 + _bf(dist) * _bf(w1p[0:1, :])
    for k in range(3):
        h1 = h1 + _bf(aux[:, 3 + k:4 + k]) * _bf(w1p[1 + k:2 + k, :])
    for k in range(3):
        h1 = h1 + _bf(gat[:, k:k + 1]) * _bf(w1p[4 + k:5 + k, :])
    h1 = h1 + _bf(d0) * _bf(w1p[7:8, :])
    h1 = h1 + _bf(d1) * _bf(w1p[8:9, :])
    h1 = h1 + _bf(d2c) * _bf(w1p[9:10, :])
    return h1, delta


def _d1_body(gat_ref, aux_ref, w1p_ref, s_ref):
    h1, _ = _h1_from_tile(gat_ref[:, :], aux_ref[:, :], w1p_ref[:, :])
    p0 = jnp.sum(h1, axis=0, keepdims=True)
    p1 = jnp.sum(h1 * h1, axis=0, keepdims=True)
    part = jnp.concatenate([p0, p1], axis=0)

    @pl.when(pl.program_id(0) == 0)
    def _():
        s_ref[:, :] = part

    @pl.when(pl.program_id(0) != 0)
    def _():
        s_ref[:, :] = s_ref[:, :] + part


def _d2_body(gat_ref, aux_ref, w1p_ref, st_ref, w2p_ref, b2p_ref,
             ymax_ref, ys_ref, *, rows, s, t_count):
    h1, delta = _h1_from_tile(gat_ref[:, :], aux_ref[:, :], w1p_ref[:, :])
    st = st_ref[:, :]
    mu = st[0:1, :] * (1.0 / t_count)
    var = st[1:2, :] * (1.0 / t_count) - mu * mu
    rs = 1.0 / jnp.sqrt(var + EPS)
    h1n = jnp.maximum((h1 - mu) * rs, 0.0)
    h2 = jnp.dot(_bf(h1n), _bf(w2p_ref[:, :]),
                 preferred_element_type=jnp.float32,
                 precision=lax.Precision.HIGHEST)
    h2 = h2 + b2p_ref[:, :]
    x = jnp.concatenate(
        [delta, gat_ref[:, 16:80], jnp.zeros((rows, 128 - CIN), jnp.float32)],
        axis=1)
    y = h2 * x
    p0 = jnp.sum(y, axis=0, keepdims=True)
    p1 = jnp.sum(y * y, axis=0, keepdims=True)
    part = jnp.concatenate([p0, p1], axis=0)

    @pl.when(pl.program_id(0) == 0)
    def _():
        ys_ref[:, :] = part

    @pl.when(pl.program_id(0) != 0)
    def _():
        ys_ref[:, :] = ys_ref[:, :] + part

    for k in range(rows // s):
        ymax_ref[k:k + 1, :] = jnp.max(y[k * s:(k + 1) * s, :], axis=0,
                                       keepdims=True)


def _e_body(ymax_ref, ys_ref, wcrp_ref, bcr_ref, out_ref, *, t_count):
    st = ys_ref[:, :]
    mu = st[0:1, :] * (1.0 / t_count)
    var = st[1:2, :] * (1.0 / t_count) - mu * mu
    rs = 1.0 / jnp.sqrt(var + EPS)
    yn = jnp.maximum((ymax_ref[:, :] - mu) * rs, 0.0)
    z = jnp.dot(_bf(yn), _bf(wcrp_ref[:, :]),
                preferred_element_type=jnp.float32,
                precision=lax.Precision.HIGHEST)
    z = z + bcr_ref[:, :]
    muz = jnp.sum(z, axis=0, keepdims=True) * (1.0 / (B * P))
    varz = jnp.sum(z * z, axis=0, keepdims=True) * (1.0 / (B * P)) - muz * muz
    out_ref[:, :] = jnp.maximum((z - muz) / jnp.sqrt(varz + EPS), 0.0)


def _rsconv_scale(gat, aux, w1p, w2p, b2p, wcrp, bcrp, s):
    t_rows = gat.shape[0]
    rows = 512
    grid = (t_rows // rows,)
    stats = pl.pallas_call(
        _d1_body,
        grid=grid,
        in_specs=[
            pl.BlockSpec((rows, D), lambda i: (i, 0)),
            pl.BlockSpec((rows, 8), lambda i: (i, 0)),
            pl.BlockSpec((16, CMID), lambda i: (0, 0)),
        ],
        out_specs=pl.BlockSpec((2, CMID), lambda i: (0, 0)),
        out_shape=jax.ShapeDtypeStruct((2, CMID), jnp.float32),
    )(gat, aux, w1p)
    ymax, ys = pl.pallas_call(
        functools.partial(_d2_body, rows=rows, s=s, t_count=float(t_rows)),
        grid=grid,
        in_specs=[
            pl.BlockSpec((rows, D), lambda i: (i, 0)),
            pl.BlockSpec((rows, 8), lambda i: (i, 0)),
            pl.BlockSpec((16, CMID), lambda i: (0, 0)),
            pl.BlockSpec((2, CMID), lambda i: (0, 0)),
            pl.BlockSpec((CMID, 128), lambda i: (0, 0)),
            pl.BlockSpec((1, 128), lambda i: (0, 0)),
        ],
        out_specs=[
            pl.BlockSpec((rows // s, 128), lambda i: (i, 0)),
            pl.BlockSpec((2, 128), lambda i: (0, 0)),
        ],
        out_shape=[
            jax.ShapeDtypeStruct((B * P, 128), jnp.float32),
            jax.ShapeDtypeStruct((2, 128), jnp.float32),
        ],
    )(gat, aux, w1p, stats, w2p, b2p)
    out = pl.pallas_call(
        functools.partial(_e_body, t_count=float(t_rows)),
        in_specs=[
            pl.BlockSpec((B * P, 128), lambda: (0, 0)),
            pl.BlockSpec((2, 128), lambda: (0, 0)),
            pl.BlockSpec((128, 128), lambda: (0, 0)),
            pl.BlockSpec((1, 128), lambda: (0, 0)),
        ],
        out_specs=pl.BlockSpec((B * P, 128), lambda: (0, 0)),
        out_shape=jax.ShapeDtypeStruct((B * P, 128), jnp.float32),
    )(ymax, ys, wcrp, bcrp)
    return out


def kernel(xyz, features, w1, b1, w2, b2, wcr, bcr):
    xs = xyz[:, :, 0]
    ys = xyz[:, :, 1]
    zs = xyz[:, :, 2]
    nx, ny, nz = _fps(xs, ys, zs)
    new_xyz = jnp.stack([nx, ny, nz], axis=-1)

    idx1, idx2 = _ballquery_sc(xs, ys, zs, nx, ny, nz)

    table = jnp.concatenate(
        [xyz, jnp.zeros((B, N, 13), jnp.float32),
         jnp.transpose(features, (0, 2, 1))], axis=2).reshape(B * N, D)
    gat = _gather_sc(table, jnp.concatenate([idx1, idx2]))
    gat1 = gat[:T1]
    gat2 = gat[T1:]

    newx_flat = new_xyz.reshape(B * P, 3)

    def make_aux(g, s):
        nrep = jnp.repeat(newx_flat, s, axis=0)
        xi = jnp.repeat(g.reshape(B * P, s, D)[:, 0, 0:3], s, axis=0)
        return jnp.concatenate(
            [nrep, xi, jnp.zeros((B * P * s, 2), jnp.float32)], axis=1)

    w1p = jnp.zeros((16, CMID), jnp.float32).at[0:10].set(w1.T).at[15].set(b1)
    w2p = jnp.zeros((CMID, 128), jnp.float32).at[:, 0:CIN].set(w2.T)
    b2p = jnp.zeros((1, 128), jnp.float32).at[:, 0:CIN].set(b2)
    wcrp = jnp.zeros((128, 128), jnp.float32).at[0:CIN].set(wcr.T)
    bcrp = bcr.reshape(1, 128)

    o1 = _rsconv_scale(gat1, make_aux(gat1, S1), w1p, w2p, b2p, wcrp, bcrp, S1)
    o2 = _rsconv_scale(gat2, make_aux(gat2, S2), w1p, w2p, b2p, wcrp, bcrp, S2)
    out = jnp.concatenate([
        o1.reshape(B, P, COUT).transpose(0, 2, 1),
        o2.reshape(B, P, COUT).transpose(0, 2, 1),
    ], axis=1)
    return (new_xyz, out)


# speedup vs baseline: 118.0537x; 1.0845x over previous
import functools

import jax
import jax.numpy as jnp
import numpy as np
from jax import lax
from jax.experimental import pallas as pl
from jax.experimental.pallas import tpu as pltpu
from jax.experimental.pallas import tpu_sc as plsc

B = 4
N = 8192
P = 1024
S1, S2 = 16, 32
CF = 64
CIN = CF + 3
CMID = 32
COUT = 128
EPS = 1e-5
R1SQ = np.float32(0.1 * 0.1)
R2SQ = np.float32(0.2 * 0.2)
D = 80
T1 = B * P * S1
T2 = B * P * S2

NC, NS = 2, 16
NW = NC * NS
PC = (B * P) // NW


def _fps_body(x_ref, y_ref, z_ref, nx_ref, ny_ref, nz_ref, dists_ref):
    x = x_ref[:, :]
    y = y_ref[:, :]
    z = z_ref[:, :]
    iota = lax.broadcasted_iota(jnp.int32, (B, N), 1)
    piota = lax.broadcasted_iota(jnp.int32, (B, P), 1)
    dists_ref[:, :] = jnp.full((B, N), 1e10, jnp.float32)
    cx0 = x[:, 0:1]
    cy0 = y[:, 0:1]
    cz0 = z[:, 0:1]
    nx_ref[:, :] = jnp.broadcast_to(cx0, (B, P))
    ny_ref[:, :] = jnp.broadcast_to(cy0, (B, P))
    nz_ref[:, :] = jnp.broadcast_to(cz0, (B, P))

    def body(i, cur):
        cx, cy, cz = cur
        dx = x - cx
        dy = y - cy
        dz = z - cz
        d = (dx * dx + dy * dy) + dz * dz
        dists = jnp.minimum(dists_ref[:, :], d)
        dists_ref[:, :] = dists
        m = jnp.max(dists, axis=1, keepdims=True)
        idx = jnp.min(jnp.where(dists == m, iota, N), axis=1, keepdims=True)
        hit = iota == idx
        ncx = jnp.sum(jnp.where(hit, x, 0.0), axis=1, keepdims=True)
        ncy = jnp.sum(jnp.where(hit, y, 0.0), axis=1, keepdims=True)
        ncz = jnp.sum(jnp.where(hit, z, 0.0), axis=1, keepdims=True)
        sel = piota == i
        nx_ref[:, :] = jnp.where(sel, ncx, nx_ref[:, :])
        ny_ref[:, :] = jnp.where(sel, ncy, ny_ref[:, :])
        nz_ref[:, :] = jnp.where(sel, ncz, nz_ref[:, :])
        return (ncx, ncy, ncz)

    lax.fori_loop(1, P, body, (cx0, cy0, cz0))


def _fps(xs, ys, zs):
    return pl.pallas_call(
        _fps_body,
        out_shape=[jax.ShapeDtypeStruct((B, P), jnp.float32)] * 3,
        scratch_shapes=[pltpu.VMEM((B, N), jnp.float32)],
    )(xs, ys, zs)


_SC_MESH = plsc.VectorSubcoreMesh(
    core_axis_name="c", subcore_axis_name="s", num_cores=NC, num_subcores=NS
)


@functools.partial(
    pl.kernel,
    out_type=[
        jax.ShapeDtypeStruct((B * P * S1,), jnp.int32),
        jax.ShapeDtypeStruct((B * P * S2,), jnp.int32),
    ],
    mesh=_SC_MESH,
    compiler_params=pltpu.CompilerParams(needs_layout_passes=False),
    scratch_types=[
        pltpu.VMEM((N,), jnp.float32),
        pltpu.VMEM((N,), jnp.float32),
        pltpu.VMEM((N,), jnp.float32),
        pltpu.VMEM((PC,), jnp.float32),
        pltpu.VMEM((PC,), jnp.float32),
        pltpu.VMEM((PC,), jnp.float32),
        pltpu.VMEM((N + 32,), jnp.int32),
        pltpu.VMEM((N + 32,), jnp.int32),
        pltpu.VMEM((N + 32,), jnp.int32),
        pltpu.VMEM((N + 32,), jnp.int32),
        pltpu.VMEM((PC * S1,), jnp.int32),
        pltpu.VMEM((PC * S2,), jnp.int32),
    ],
)
def _ballquery_sc(xs_hbm, ys_hbm, zs_hbm, nxs_hbm, nys_hbm, nzs_hbm,
                  out1_hbm, out2_hbm, xv, yv, zv, nxv, nyv, nzv,
                  ca1, ca2, cb1, cb2, o1, o2):
    wid = lax.axis_index("s") * NC + lax.axis_index("c")
    tiles_per_b = NW // B
    b = wid // tiles_per_b
    p0 = (wid % tiles_per_b) * PC
    pltpu.sync_copy(xs_hbm.at[b], xv)
    pltpu.sync_copy(ys_hbm.at[b], yv)
    pltpu.sync_copy(zs_hbm.at[b], zv)
    pltpu.sync_copy(nxs_hbm.at[b, pl.ds(p0, PC)], nxv)
    pltpu.sync_copy(nys_hbm.at[b, pl.ds(p0, PC)], nyv)
    pltpu.sync_copy(nzs_hbm.at[b, pl.ds(p0, PC)], nzv)
    lane = lax.iota(jnp.int32, 16)
    gbase = b * N

    HP = PC // 2

    def per_pair(i, _):
        ia, ib = i, i + HP
        cxa = plsc.load_gather(nxv, [jnp.full((16,), ia, jnp.int32)])
        cya = plsc.load_gather(nyv, [jnp.full((16,), ia, jnp.int32)])
        cza = plsc.load_gather(nzv, [jnp.full((16,), ia, jnp.int32)])
        cxb = plsc.load_gather(nxv, [jnp.full((16,), ib, jnp.int32)])
        cyb = plsc.load_gather(nyv, [jnp.full((16,), ib, jnp.int32)])
        czb = plsc.load_gather(nzv, [jnp.full((16,), ib, jnp.int32)])

        def cond(state):
            c, a1, a2, b1, b2 = state
            return (c < N // 32) & ((a1 < S1) | (a2 < S2)
                                    | (b1 < S1) | (b2 < S2))

        def body(state):
            c, a1, a2, b1, b2 = state
            for u in range(2):
                off = c * 32 + u * 16
                xc = xv[pl.ds(off, 16)]
                yc = yv[pl.ds(off, 16)]
                zc = zv[pl.ds(off, 16)]
                jv = off + lane
                dxa = cxa - xc
                dya = cya - yc
                dza = cza - zc
                d2a = (dxa * dxa + dya * dya) + dza * dza
                dxb = cxb - xc
                dyb = cyb - yc
                dzb = czb - zc
                d2b = (dxb * dxb + dyb * dyb) + dzb * dzb
                ma1 = d2a < R1SQ
                ma2 = d2a < R2SQ
                mb1 = d2b < R1SQ
                mb2 = d2b < R2SQ
                plsc.store_compressed(ca1.at[pl.ds(a1, 16)], jv, mask=ma1)
                plsc.store_compressed(ca2.at[pl.ds(a2, 16)], jv, mask=ma2)
                plsc.store_compressed(cb1.at[pl.ds(b1, 16)], jv, mask=mb1)
                plsc.store_compressed(cb2.at[pl.ds(b2, 16)], jv, mask=mb2)
                a1 = a1 + plsc.all_reduce_population_count(ma1)[0]
                a2 = a2 + plsc.all_reduce_population_count(ma2)[0]
                b1 = b1 + plsc.all_reduce_population_count(mb1)[0]
                b2 = b2 + plsc.all_reduce_population_count(mb2)[0]
            return (c + 1, a1, a2, b1, b2)

        _, a1, a2, b1, b2 = lax.while_loop(cond, body, (0, 0, 0, 0, 0))

        true16 = lane < 16
        for (cand1, cand2, n1, n2, ci) in ((ca1, ca2, a1, a2, ia),
                                           (cb1, cb2, b1, b2, ib)):
            f1 = jnp.full((16,), cand1[pl.ds(0, 16)][0], jnp.int32)
            plsc.store_compressed(cand1.at[pl.ds(jnp.minimum(n1, S1), 16)],
                                  f1, mask=true16)
            o1[pl.ds(ci * S1, 16)] = cand1[pl.ds(0, 16)] + gbase
            f2 = jnp.full((16,), cand2[pl.ds(0, 16)][0], jnp.int32)
            plsc.store_compressed(cand2.at[pl.ds(jnp.minimum(n2, S2), 16)],
                                  f2, mask=true16)
            for k in range(S2 // 16):
                o2[pl.ds(ci * S2 + k * 16, 16)] = (cand2[pl.ds(k * 16, 16)]
                                                   + gbase)
        return 0

    lax.fori_loop(0, HP, per_pair, 0)
    pltpu.sync_copy(o1, out1_hbm.at[pl.ds(wid * PC * S1, PC * S1)])
    pltpu.sync_copy(o2, out2_hbm.at[pl.ds(wid * PC * S2, PC * S2)])


TT = T1 + T2
GCHUNK = 128


@functools.partial(
    pl.kernel,
    out_type=jax.ShapeDtypeStruct((TT, D), jnp.float32),
    mesh=_SC_MESH,
    compiler_params=pltpu.CompilerParams(
        needs_layout_passes=False, use_tc_tiling_on_sc=False),
    scratch_types=[
        pltpu.VMEM((GCHUNK,), jnp.int32),
        pltpu.VMEM((GCHUNK, D), jnp.float32),
        pltpu.SemaphoreType.DMA,
    ],
)
def _gather_sc(table_hbm, idx_hbm, out_hbm, idx_v, rows_v, sem):
    wid = lax.axis_index("s") * NC + lax.axis_index("c")
    per_w = TT // NW
    base = wid * per_w

    def chunk(k, _):
        off = base + k * GCHUNK
        pltpu.sync_copy(idx_hbm.at[pl.ds(off, GCHUNK)], idx_v)
        pltpu.async_copy(table_hbm.at[idx_v], rows_v, sem).wait()
        pltpu.sync_copy(rows_v, out_hbm.at[pl.ds(off, GCHUNK)])
        return 0

    lax.fori_loop(0, per_w // GCHUNK, chunk, 0)


def _bf(v):
    return v.astype(jnp.bfloat16).astype(jnp.float32)


def _h1_from_tile(gat, aux, w1p):
    raw = gat[:, 0:3]
    delta = raw - aux[:, 0:3]
    d0 = delta[:, 0:1]
    d1 = delta[:, 1:2]
    d2c = delta[:, 2:3]
    dist = jnp.sqrt((d0 * d0 + d1 * d1) + d2c * d2c + 1e-12)
    h1 = w1p[15:16, :]
    h1 = h1 + _bf(dist) * _bf(w1p[0:1, :])
    for k in range(3):
        h1 = h1 + _bf(aux[:, 3 + k:4 + k]) * _bf(w1p[1 + k:2 + k, :])
    for k in range(3):
        h1 = h1 + _bf(gat[:, k:k + 1]) * _bf(w1p[4 + k:5 + k, :])
    h1 = h1 + _bf(d0) * _bf(w1p[7:8, :])
    h1 = h1 + _bf(d1) * _bf(w1p[8:9, :])
    h1 = h1 + _bf(d2c) * _bf(w1p[9:10, :])
    return h1, delta


def _d1_body(gat_ref, aux_ref, w1p_ref, s_ref):
    h1, _ = _h1_from_tile(gat_ref[:, :], aux_ref[:, :], w1p_ref[:, :])
    p0 = jnp.sum(h1, axis=0, keepdims=True)
    p1 = jnp.sum(h1 * h1, axis=0, keepdims=True)
    part = jnp.concatenate([p0, p1], axis=0)

    @pl.when(pl.program_id(0) == 0)
    def _():
        s_ref[:, :] = part

    @pl.when(pl.program_id(0) != 0)
    def _():
        s_ref[:, :] = s_ref[:, :] + part


def _d2_body(gat_ref, aux_ref, w1p_ref, st_ref, w2p_ref, b2p_ref,
             ymax_ref, ys_ref, *, rows, s, t_count):
    h1, delta = _h1_from_tile(gat_ref[:, :], aux_ref[:, :], w1p_ref[:, :])
    st = st_ref[:, :]
    mu = st[0:1, :] * (1.0 / t_count)
    var = st[1:2, :] * (1.0 / t_count) - mu * mu
    rs = 1.0 / jnp.sqrt(var + EPS)
    h1n = jnp.maximum((h1 - mu) * rs, 0.0)
    h2 = jnp.dot(_bf(h1n), _bf(w2p_ref[:, :]),
                 preferred_element_type=jnp.float32,
                 precision=lax.Precision.HIGHEST)
    h2 = h2 + b2p_ref[:, :]
    x = jnp.concatenate(
        [delta, gat_ref[:, 16:80], jnp.zeros((rows, 128 - CIN), jnp.float32)],
        axis=1)
    y = h2 * x
    p0 = jnp.sum(y, axis=0, keepdims=True)
    p1 = jnp.sum(y * y, axis=0, keepdims=True)
    part = jnp.concatenate([p0, p1], axis=0)

    @pl.when(pl.program_id(0) == 0)
    def _():
        ys_ref[:, :] = part

    @pl.when(pl.program_id(0) != 0)
    def _():
        ys_ref[:, :] = ys_ref[:, :] + part

    for k in range(rows // s):
        ymax_ref[k:k + 1, :] = jnp.max(y[k * s:(k + 1) * s, :], axis=0,
                                       keepdims=True)


def _e_body(ymax_ref, ys_ref, wcrp_ref, bcr_ref, out_ref, *, t_count):
    st = ys_ref[:, :]
    mu = st[0:1, :] * (1.0 / t_count)
    var = st[1:2, :] * (1.0 / t_count) - mu * mu
    rs = 1.0 / jnp.sqrt(var + EPS)
    yn = jnp.maximum((ymax_ref[:, :] - mu) * rs, 0.0)
    z = jnp.dot(_bf(yn), _bf(wcrp_ref[:, :]),
                preferred_element_type=jnp.float32,
                precision=lax.Precision.HIGHEST)
    z = z + bcr_ref[:, :]
    muz = jnp.sum(z, axis=0, keepdims=True) * (1.0 / (B * P))
    varz = jnp.sum(z * z, axis=0, keepdims=True) * (1.0 / (B * P)) - muz * muz
    out_ref[:, :] = jnp.maximum((z - muz) / jnp.sqrt(varz + EPS), 0.0)


def _rsconv_scale(gat, aux, w1p, w2p, b2p, wcrp, bcrp, s):
    t_rows = gat.shape[0]
    rows = 512
    grid = (t_rows // rows,)
    stats = pl.pallas_call(
        _d1_body,
        grid=grid,
        in_specs=[
            pl.BlockSpec((rows, D), lambda i: (i, 0)),
            pl.BlockSpec((rows, 8), lambda i: (i, 0)),
            pl.BlockSpec((16, CMID), lambda i: (0, 0)),
        ],
        out_specs=pl.BlockSpec((2, CMID), lambda i: (0, 0)),
        out_shape=jax.ShapeDtypeStruct((2, CMID), jnp.float32),
    )(gat, aux, w1p)
    ymax, ys = pl.pallas_call(
        functools.partial(_d2_body, rows=rows, s=s, t_count=float(t_rows)),
        grid=grid,
        in_specs=[
            pl.BlockSpec((rows, D), lambda i: (i, 0)),
            pl.BlockSpec((rows, 8), lambda i: (i, 0)),
            pl.BlockSpec((16, CMID), lambda i: (0, 0)),
            pl.BlockSpec((2, CMID), lambda i: (0, 0)),
            pl.BlockSpec((CMID, 128), lambda i: (0, 0)),
            pl.BlockSpec((1, 128), lambda i: (0, 0)),
        ],
        out_specs=[
            pl.BlockSpec((rows // s, 128), lambda i: (i, 0)),
            pl.BlockSpec((2, 128), lambda i: (0, 0)),
        ],
        out_shape=[
            jax.ShapeDtypeStruct((B * P, 128), jnp.float32),
            jax.ShapeDtypeStruct((2, 128), jnp.float32),
        ],
    )(gat, aux, w1p, stats, w2p, b2p)
    out = pl.pallas_call(
        functools.partial(_e_body, t_count=float(t_rows)),
        in_specs=[
            pl.BlockSpec((B * P, 128), lambda: (0, 0)),
            pl.BlockSpec((2, 128), lambda: (0, 0)),
            pl.BlockSpec((128, 128), lambda: (0, 0)),
            pl.BlockSpec((1, 128), lambda: (0, 0)),
        ],
        out_specs=pl.BlockSpec((B * P, 128), lambda: (0, 0)),
        out_shape=jax.ShapeDtypeStruct((B * P, 128), jnp.float32),
    )(ymax, ys, wcrp, bcrp)
    return out


def kernel(xyz, features, w1, b1, w2, b2, wcr, bcr):
    xs = xyz[:, :, 0]
    ys = xyz[:, :, 1]
    zs = xyz[:, :, 2]
    nx, ny, nz = _fps(xs, ys, zs)
    new_xyz = jnp.stack([nx, ny, nz], axis=-1)

    idx1, idx2 = _ballquery_sc(xs, ys, zs, nx, ny, nz)

    table = jnp.concatenate(
        [xyz, jnp.zeros((B, N, 13), jnp.float32),
         jnp.transpose(features, (0, 2, 1))], axis=2).reshape(B * N, D)
    gat = _gather_sc(table, jnp.concatenate([idx1, idx2]))
    gat1 = gat[:T1]
    gat2 = gat[T1:]

    newx_flat = new_xyz.reshape(B * P, 3)

    def make_aux(g, s):
        nrep = jnp.repeat(newx_flat, s, axis=0)
        xi = jnp.repeat(g.reshape(B * P, s, D)[:, 0, 0:3], s, axis=0)
        return jnp.concatenate(
            [nrep, xi, jnp.zeros((B * P * s, 2), jnp.float32)], axis=1)

    w1p = jnp.zeros((16, CMID), jnp.float32).at[0:10].set(w1.T).at[15].set(b1)
    w2p = jnp.zeros((CMID, 128), jnp.float32).at[:, 0:CIN].set(w2.T)
    b2p = jnp.zeros((1, 128), jnp.float32).at[:, 0:CIN].set(b2)
    wcrp = jnp.zeros((128, 128), jnp.float32).at[0:CIN].set(wcr.T)
    bcrp = bcr.reshape(1, 128)

    o1 = _rsconv_scale(gat1, make_aux(gat1, S1), w1p, w2p, b2p, wcrp, bcrp, S1)
    o2 = _rsconv_scale(gat2, make_aux(gat2, S2), w1p, w2p, b2p, wcrp, bcrp, S2)
    out = jnp.concatenate([
        o1.reshape(B, P, COUT).transpose(0, 2, 1),
        o2.reshape(B, P, COUT).transpose(0, 2, 1),
    ], axis=1)
    return (new_xyz, out)


# speedup vs baseline: 120.5092x; 1.0208x over previous
import functools

import jax
import jax.numpy as jnp
import numpy as np
from jax import lax
from jax.experimental import pallas as pl
from jax.experimental.pallas import tpu as pltpu
from jax.experimental.pallas import tpu_sc as plsc

B = 4
N = 8192
P = 1024
S1, S2 = 16, 32
CF = 64
CIN = CF + 3
CMID = 32
COUT = 128
EPS = 1e-5
R1SQ = np.float32(0.1 * 0.1)
R2SQ = np.float32(0.2 * 0.2)
D = 80
T1 = B * P * S1
T2 = B * P * S2

NC, NS = 2, 16
NW = NC * NS
PC = (B * P) // NW


def _fps_body(x_ref, y_ref, z_ref, nx_ref, ny_ref, nz_ref, dists_ref):
    x = x_ref[:, :]
    y = y_ref[:, :]
    z = z_ref[:, :]
    iota = lax.broadcasted_iota(jnp.int32, (B, N), 1)
    piota = lax.broadcasted_iota(jnp.int32, (B, P), 1)
    dists_ref[:, :] = jnp.full((B, N), 1e10, jnp.float32)
    cx0 = x[:, 0:1]
    cy0 = y[:, 0:1]
    cz0 = z[:, 0:1]
    nx_ref[:, :] = jnp.broadcast_to(cx0, (B, P))
    ny_ref[:, :] = jnp.broadcast_to(cy0, (B, P))
    nz_ref[:, :] = jnp.broadcast_to(cz0, (B, P))

    def body(i, cur):
        cx, cy, cz = cur
        dx = x - cx
        dy = y - cy
        dz = z - cz
        d = (dx * dx + dy * dy) + dz * dz
        dists = jnp.minimum(dists_ref[:, :], d)
        dists_ref[:, :] = dists
        m = jnp.max(dists, axis=1, keepdims=True)
        idx = jnp.min(jnp.where(dists == m, iota, N), axis=1, keepdims=True)
        hit = iota == idx
        ncx = jnp.sum(jnp.where(hit, x, 0.0), axis=1, keepdims=True)
        ncy = jnp.sum(jnp.where(hit, y, 0.0), axis=1, keepdims=True)
        ncz = jnp.sum(jnp.where(hit, z, 0.0), axis=1, keepdims=True)
        sel = piota == i
        nx_ref[:, :] = jnp.where(sel, ncx, nx_ref[:, :])
        ny_ref[:, :] = jnp.where(sel, ncy, ny_ref[:, :])
        nz_ref[:, :] = jnp.where(sel, ncz, nz_ref[:, :])
        return (ncx, ncy, ncz)

    lax.fori_loop(1, P, body, (cx0, cy0, cz0))


def _fps(xs, ys, zs):
    return pl.pallas_call(
        _fps_body,
        out_shape=[jax.ShapeDtypeStruct((B, P), jnp.float32)] * 3,
        scratch_shapes=[pltpu.VMEM((B, N), jnp.float32)],
    )(xs, ys, zs)


_SC_MESH = plsc.VectorSubcoreMesh(
    core_axis_name="c", subcore_axis_name="s", num_cores=NC, num_subcores=NS
)


@functools.partial(
    pl.kernel,
    out_type=[
        jax.ShapeDtypeStruct((B * P * S1,), jnp.int32),
        jax.ShapeDtypeStruct((B * P * S2,), jnp.int32),
    ],
    mesh=_SC_MESH,
    compiler_params=pltpu.CompilerParams(needs_layout_passes=False),
    scratch_types=[
        pltpu.VMEM((N,), jnp.float32),
        pltpu.VMEM((N,), jnp.float32),
        pltpu.VMEM((N,), jnp.float32),
        pltpu.VMEM((PC,), jnp.float32),
        pltpu.VMEM((PC,), jnp.float32),
        pltpu.VMEM((PC,), jnp.float32),
        pltpu.VMEM((S1 + 16,), jnp.int32),
        pltpu.VMEM((N + 32,), jnp.int32),
        pltpu.VMEM((S1 + 16,), jnp.int32),
        pltpu.VMEM((N + 32,), jnp.int32),
        pltpu.VMEM((PC * S1,), jnp.int32),
        pltpu.VMEM((PC * S2,), jnp.int32),
    ],
)
def _ballquery_sc(xs_hbm, ys_hbm, zs_hbm, nxs_hbm, nys_hbm, nzs_hbm,
                  out1_hbm, out2_hbm, xv, yv, zv, nxv, nyv, nzv,
                  ca1, ca2, cb1, cb2, o1, o2):
    wid = lax.axis_index("s") * NC + lax.axis_index("c")
    tiles_per_b = NW // B
    b = wid // tiles_per_b
    p0 = (wid % tiles_per_b) * PC
    pltpu.sync_copy(xs_hbm.at[b], xv)
    pltpu.sync_copy(ys_hbm.at[b], yv)
    pltpu.sync_copy(zs_hbm.at[b], zv)
    pltpu.sync_copy(nxs_hbm.at[b, pl.ds(p0, PC)], nxv)
    pltpu.sync_copy(nys_hbm.at[b, pl.ds(p0, PC)], nyv)
    pltpu.sync_copy(nzs_hbm.at[b, pl.ds(p0, PC)], nzv)
    lane = lax.iota(jnp.int32, 16)
    gbase = b * N

    HP = PC // 2

    def per_pair(i, _):
        ia, ib = i, i + HP
        cxa = plsc.load_gather(nxv, [jnp.full((16,), ia, jnp.int32)])
        cya = plsc.load_gather(nyv, [jnp.full((16,), ia, jnp.int32)])
        cza = plsc.load_gather(nzv, [jnp.full((16,), ia, jnp.int32)])
        cxb = plsc.load_gather(nxv, [jnp.full((16,), ib, jnp.int32)])
        cyb = plsc.load_gather(nyv, [jnp.full((16,), ib, jnp.int32)])
        czb = plsc.load_gather(nzv, [jnp.full((16,), ib, jnp.int32)])

        def cond(state):
            c, a1, a2, b1, b2 = state
            return (c < N // 32) & ((a1 < S1) | (a2 < S2)
                                    | (b1 < S1) | (b2 < S2))

        def body(state):
            c, a1, a2, b1, b2 = state
            for u in range(2):
                off = c * 32 + u * 16
                xc = xv[pl.ds(off, 16)]
                yc = yv[pl.ds(off, 16)]
                zc = zv[pl.ds(off, 16)]
                jv = off + lane
                dxa = cxa - xc
                dya = cya - yc
                dza = cza - zc
                d2a = (dxa * dxa + dya * dya) + dza * dza
                dxb = cxb - xc
                dyb = cyb - yc
                dzb = czb - zc
                d2b = (dxb * dxb + dyb * dyb) + dzb * dzb
                ma1 = d2a < R1SQ
                ma2 = d2a < R2SQ
                mb1 = d2b < R1SQ
                mb2 = d2b < R2SQ
                plsc.store_compressed(ca2.at[pl.ds(a2, 16)], jv, mask=ma2)
                plsc.store_compressed(cb2.at[pl.ds(b2, 16)], jv, mask=mb2)
                a1 = a1 + plsc.all_reduce_population_count(ma1)[0]
                a2 = a2 + plsc.all_reduce_population_count(ma2)[0]
                b1 = b1 + plsc.all_reduce_population_count(mb1)[0]
                b2 = b2 + plsc.all_reduce_population_count(mb2)[0]
            return (c + 1, a1, a2, b1, b2)

        _, a1, a2, b1, b2 = lax.while_loop(cond, body, (0, 0, 0, 0, 0))

        true16 = lane < 16
        for (cand1, cand2, cxc, cyc, czc, n2, ci) in (
                (ca1, ca2, cxa, cya, cza, a2, ia),
                (cb1, cb2, cxb, cyb, czb, b2, ib)):
            f2 = jnp.full((16,), cand2[pl.ds(0, 16)][0], jnp.int32)
            plsc.store_compressed(cand2.at[pl.ds(n2, 16)], f2, mask=true16)
            plsc.store_compressed(cand2.at[pl.ds(n2 + 16, 16)], f2,
                                  mask=true16)

            def pcond(s, _n2=n2):
                k, n1 = s
                return (k * 16 < _n2) & (n1 < S1)

            def pbody(s, _c1=cand1, _c2=cand2, _cx=cxc, _cy=cyc, _cz=czc):
                k, n1 = s
                jc = _c2[pl.ds(k * 16, 16)]
                dx = _cx - plsc.load_gather(xv, [jc])
                dy = _cy - plsc.load_gather(yv, [jc])
                dz = _cz - plsc.load_gather(zv, [jc])
                d2 = (dx * dx + dy * dy) + dz * dz
                m1 = d2 < R1SQ
                plsc.store_compressed(_c1.at[pl.ds(n1, 16)], jc, mask=m1)
                return (k + 1, n1 + plsc.all_reduce_population_count(m1)[0])

            _, n1 = lax.while_loop(pcond, pbody, (0, 0))

            f1 = jnp.full((16,), cand1[pl.ds(0, 16)][0], jnp.int32)
            plsc.store_compressed(cand1.at[pl.ds(jnp.minimum(n1, S1), 16)],
                                  f1, mask=true16)
            o1[pl.ds(ci * S1, 16)] = cand1[pl.ds(0, 16)] + gbase
            for k in range(S2 // 16):
                o2[pl.ds(ci * S2 + k * 16, 16)] = (cand2[pl.ds(k * 16, 16)]
                                                   + gbase)
        return 0

    lax.fori_loop(0, HP, per_pair, 0)
    pltpu.sync_copy(o1, out1_hbm.at[pl.ds(wid * PC * S1, PC * S1)])
    pltpu.sync_copy(o2, out2_hbm.at[pl.ds(wid * PC * S2, PC * S2)])


TT = T1 + T2
GCHUNK = 128


@functools.partial(
    pl.kernel,
    out_type=jax.ShapeDtypeStruct((TT, D), jnp.float32),
    mesh=_SC_MESH,
    compiler_params=pltpu.CompilerParams(
        needs_layout_passes=False, use_tc_tiling_on_sc=False),
    scratch_types=[
        pltpu.VMEM((GCHUNK,), jnp.int32),
        pltpu.VMEM((GCHUNK, D), jnp.float32),
        pltpu.SemaphoreType.DMA,
    ],
)
def _gather_sc(table_hbm, idx_hbm, out_hbm, idx_v, rows_v, sem):
    wid = lax.axis_index("s") * NC + lax.axis_index("c")
    per_w = TT // NW
    base = wid * per_w

    def chunk(k, _):
        off = base + k * GCHUNK
        pltpu.sync_copy(idx_hbm.at[pl.ds(off, GCHUNK)], idx_v)
        pltpu.async_copy(table_hbm.at[idx_v], rows_v, sem).wait()
        pltpu.sync_copy(rows_v, out_hbm.at[pl.ds(off, GCHUNK)])
        return 0

    lax.fori_loop(0, per_w // GCHUNK, chunk, 0)


def _bf(v):
    return v.astype(jnp.bfloat16).astype(jnp.float32)


def _h1_from_tile(gat, aux, w1p):
    raw = gat[:, 0:3]
    delta = raw - aux[:, 0:3]
    d0 = delta[:, 0:1]
    d1 = delta[:, 1:2]
    d2c = delta[:, 2:3]
    dist = jnp.sqrt((d0 * d0 + d1 * d1) + d2c * d2c + 1e-12)
    h1 = w1p[15:16, :]
    h1 = h1 + _bf(dist) * _bf(w1p[0:1, :])
    for k in range(3):
        h1 = h1 + _bf(aux[:, 3 + k:4 + k]) * _bf(w1p[1 + k:2 + k, :])
    for k in range(3):
        h1 = h1 + _bf(gat[:, k:k + 1]) * _bf(w1p[4 + k:5 + k, :])
    h1 = h1 + _bf(d0) * _bf(w1p[7:8, :])
    h1 = h1 + _bf(d1) * _bf(w1p[8:9, :])
    h1 = h1 + _bf(d2c) * _bf(w1p[9:10, :])
    return h1, delta


def _d1_body(gat_ref, aux_ref, w1p_ref, s_ref):
    h1, _ = _h1_from_tile(gat_ref[:, :], aux_ref[:, :], w1p_ref[:, :])
    p0 = jnp.sum(h1, axis=0, keepdims=True)
    p1 = jnp.sum(h1 * h1, axis=0, keepdims=True)
    part = jnp.concatenate([p0, p1], axis=0)

    @pl.when(pl.program_id(0) == 0)
    def _():
        s_ref[:, :] = part

    @pl.when(pl.program_id(0) != 0)
    def _():
        s_ref[:, :] = s_ref[:, :] + part


def _d2_body(gat_ref, aux_ref, w1p_ref, st_ref, w2p_ref, b2p_ref,
             ymax_ref, ys_ref, *, rows, s, t_count):
    h1, delta = _h1_from_tile(gat_ref[:, :], aux_ref[:, :], w1p_ref[:, :])
    st = st_ref[:, :]
    mu = st[0:1, :] * (1.0 / t_count)
    var = st[1:2, :] * (1.0 / t_count) - mu * mu
    rs = 1.0 / jnp.sqrt(var + EPS)
    h1n = jnp.maximum((h1 - mu) * rs, 0.0)
    h2 = jnp.dot(_bf(h1n), _bf(w2p_ref[:, :]),
                 preferred_element_type=jnp.float32,
                 precision=lax.Precision.HIGHEST)
    h2 = h2 + b2p_ref[:, :]
    x = jnp.concatenate(
        [delta, gat_ref[:, 16:80], jnp.zeros((rows, 128 - CIN), jnp.float32)],
        axis=1)
    y = h2 * x
    p0 = jnp.sum(y, axis=0, keepdims=True)
    p1 = jnp.sum(y * y, axis=0, keepdims=True)
    part = jnp.concatenate([p0, p1], axis=0)

    @pl.when(pl.program_id(0) == 0)
    def _():
        ys_ref[:, :] = part

    @pl.when(pl.program_id(0) != 0)
    def _():
        ys_ref[:, :] = ys_ref[:, :] + part

    for k in range(rows // s):
        ymax_ref[k:k + 1, :] = jnp.max(y[k * s:(k + 1) * s, :], axis=0,
                                       keepdims=True)


def _e_body(ymax_ref, ys_ref, wcrp_ref, bcr_ref, out_ref, *, t_count):
    st = ys_ref[:, :]
    mu = st[0:1, :] * (1.0 / t_count)
    var = st[1:2, :] * (1.0 / t_count) - mu * mu
    rs = 1.0 / jnp.sqrt(var + EPS)
    yn = jnp.maximum((ymax_ref[:, :] - mu) * rs, 0.0)
    z = jnp.dot(_bf(yn), _bf(wcrp_ref[:, :]),
                preferred_element_type=jnp.float32,
                precision=lax.Precision.HIGHEST)
    z = z + bcr_ref[:, :]
    muz = jnp.sum(z, axis=0, keepdims=True) * (1.0 / (B * P))
    varz = jnp.sum(z * z, axis=0, keepdims=True) * (1.0 / (B * P)) - muz * muz
    out_ref[:, :] = jnp.maximum((z - muz) / jnp.sqrt(varz + EPS), 0.0)


def _rsconv_scale(gat, aux, w1p, w2p, b2p, wcrp, bcrp, s):
    t_rows = gat.shape[0]
    rows = 512
    grid = (t_rows // rows,)
    stats = pl.pallas_call(
        _d1_body,
        grid=grid,
        in_specs=[
            pl.BlockSpec((rows, D), lambda i: (i, 0)),
            pl.BlockSpec((rows, 8), lambda i: (i, 0)),
            pl.BlockSpec((16, CMID), lambda i: (0, 0)),
        ],
        out_specs=pl.BlockSpec((2, CMID), lambda i: (0, 0)),
        out_shape=jax.ShapeDtypeStruct((2, CMID), jnp.float32),
    )(gat, aux, w1p)
    ymax, ys = pl.pallas_call(
        functools.partial(_d2_body, rows=rows, s=s, t_count=float(t_rows)),
        grid=grid,
        in_specs=[
            pl.BlockSpec((rows, D), lambda i: (i, 0)),
            pl.BlockSpec((rows, 8), lambda i: (i, 0)),
            pl.BlockSpec((16, CMID), lambda i: (0, 0)),
            pl.BlockSpec((2, CMID), lambda i: (0, 0)),
            pl.BlockSpec((CMID, 128), lambda i: (0, 0)),
            pl.BlockSpec((1, 128), lambda i: (0, 0)),
        ],
        out_specs=[
            pl.BlockSpec((rows // s, 128), lambda i: (i, 0)),
            pl.BlockSpec((2, 128), lambda i: (0, 0)),
        ],
        out_shape=[
            jax.ShapeDtypeStruct((B * P, 128), jnp.float32),
            jax.ShapeDtypeStruct((2, 128), jnp.float32),
        ],
    )(gat, aux, w1p, stats, w2p, b2p)
    out = pl.pallas_call(
        functools.partial(_e_body, t_count=float(t_rows)),
        in_specs=[
            pl.BlockSpec((B * P, 128), lambda: (0, 0)),
            pl.BlockSpec((2, 128), lambda: (0, 0)),
            pl.BlockSpec((128, 128), lambda: (0, 0)),
            pl.BlockSpec((1, 128), lambda: (0, 0)),
        ],
        out_specs=pl.BlockSpec((B * P, 128), lambda: (0, 0)),
        out_shape=jax.ShapeDtypeStruct((B * P, 128), jnp.float32),
    )(ymax, ys, wcrp, bcrp)
    return out


def kernel(xyz, features, w1, b1, w2, b2, wcr, bcr):
    xs = xyz[:, :, 0]
    ys = xyz[:, :, 1]
    zs = xyz[:, :, 2]
    nx, ny, nz = _fps(xs, ys, zs)
    new_xyz = jnp.stack([nx, ny, nz], axis=-1)

    idx1, idx2 = _ballquery_sc(xs, ys, zs, nx, ny, nz)

    table = jnp.concatenate(
        [xyz, jnp.zeros((B, N, 13), jnp.float32),
         jnp.transpose(features, (0, 2, 1))], axis=2).reshape(B * N, D)
    gat = _gather_sc(table, jnp.concatenate([idx1, idx2]))
    gat1 = gat[:T1]
    gat2 = gat[T1:]

    newx_flat = new_xyz.reshape(B * P, 3)

    def make_aux(g, s):
        nrep = jnp.repeat(newx_flat, s, axis=0)
        xi = jnp.repeat(g.reshape(B * P, s, D)[:, 0, 0:3], s, axis=0)
        return jnp.concatenate(
            [nrep, xi, jnp.zeros((B * P * s, 2), jnp.float32)], axis=1)

    w1p = jnp.zeros((16, CMID), jnp.float32).at[0:10].set(w1.T).at[15].set(b1)
    w2p = jnp.zeros((CMID, 128), jnp.float32).at[:, 0:CIN].set(w2.T)
    b2p = jnp.zeros((1, 128), jnp.float32).at[:, 0:CIN].set(b2)
    wcrp = jnp.zeros((128, 128), jnp.float32).at[0:CIN].set(wcr.T)
    bcrp = bcr.reshape(1, 128)

    o1 = _rsconv_scale(gat1, make_aux(gat1, S1), w1p, w2p, b2p, wcrp, bcrp, S1)
    o2 = _rsconv_scale(gat2, make_aux(gat2, S2), w1p, w2p, b2p, wcrp, bcrp, S2)
    out = jnp.concatenate([
        o1.reshape(B, P, COUT).transpose(0, 2, 1),
        o2.reshape(B, P, COUT).transpose(0, 2, 1),
    ], axis=1)
    return (new_xyz, out)


# speedup vs baseline: 122.8886x; 1.0197x over previous
import functools

import jax
import jax.numpy as jnp
import numpy as np
from jax import lax
from jax.experimental import pallas as pl
from jax.experimental.pallas import tpu as pltpu
from jax.experimental.pallas import tpu_sc as plsc

B = 4
N = 8192
P = 1024
S1, S2 = 16, 32
CF = 64
CIN = CF + 3
CMID = 32
COUT = 128
EPS = 1e-5
R1SQ = np.float32(0.1 * 0.1)
R2SQ = np.float32(0.2 * 0.2)
D = 80
T1 = B * P * S1
T2 = B * P * S2

NC, NS = 2, 16
NW = NC * NS
PC = (B * P) // NW


def _fps_body(x_ref, y_ref, z_ref, nx_ref, ny_ref, nz_ref, dists_ref):
    x = x_ref[:, :]
    y = y_ref[:, :]
    z = z_ref[:, :]
    iota = lax.broadcasted_iota(jnp.int32, (B, N), 1)
    piota = lax.broadcasted_iota(jnp.int32, (B, P), 1)
    dists_ref[:, :] = jnp.full((B, N), 1e10, jnp.float32)
    cx0 = x[:, 0:1]
    cy0 = y[:, 0:1]
    cz0 = z[:, 0:1]
    nx_ref[:, :] = jnp.broadcast_to(cx0, (B, P))
    ny_ref[:, :] = jnp.broadcast_to(cy0, (B, P))
    nz_ref[:, :] = jnp.broadcast_to(cz0, (B, P))

    def body(i, cur):
        cx, cy, cz = cur
        dx = x - cx
        dy = y - cy
        dz = z - cz
        d = (dx * dx + dy * dy) + dz * dz
        dists = jnp.minimum(dists_ref[:, :], d)
        dists_ref[:, :] = dists
        m = jnp.max(dists, axis=1, keepdims=True)
        idx = jnp.min(jnp.where(dists == m, iota, N), axis=1, keepdims=True)
        hit = iota == idx
        ncx = jnp.sum(jnp.where(hit, x, 0.0), axis=1, keepdims=True)
        ncy = jnp.sum(jnp.where(hit, y, 0.0), axis=1, keepdims=True)
        ncz = jnp.sum(jnp.where(hit, z, 0.0), axis=1, keepdims=True)
        sel = piota == i
        nx_ref[:, :] = jnp.where(sel, ncx, nx_ref[:, :])
        ny_ref[:, :] = jnp.where(sel, ncy, ny_ref[:, :])
        nz_ref[:, :] = jnp.where(sel, ncz, nz_ref[:, :])
        return (ncx, ncy, ncz)

    lax.fori_loop(1, P, body, (cx0, cy0, cz0))


def _fps(xs, ys, zs):
    return pl.pallas_call(
        _fps_body,
        out_shape=[jax.ShapeDtypeStruct((B, P), jnp.float32)] * 3,
        scratch_shapes=[pltpu.VMEM((B, N), jnp.float32)],
    )(xs, ys, zs)


_SC_MESH = plsc.VectorSubcoreMesh(
    core_axis_name="c", subcore_axis_name="s", num_cores=NC, num_subcores=NS
)


@functools.partial(
    pl.kernel,
    out_type=[
        jax.ShapeDtypeStruct((B * P * S1,), jnp.int32),
        jax.ShapeDtypeStruct((B * P * S2,), jnp.int32),
    ],
    mesh=_SC_MESH,
    compiler_params=pltpu.CompilerParams(needs_layout_passes=False),
    scratch_types=[
        pltpu.VMEM((N,), jnp.float32),
        pltpu.VMEM((N,), jnp.float32),
        pltpu.VMEM((N,), jnp.float32),
        pltpu.VMEM((PC,), jnp.float32),
        pltpu.VMEM((PC,), jnp.float32),
        pltpu.VMEM((PC,), jnp.float32),
        pltpu.VMEM((S1 + 16,), jnp.int32),
        pltpu.VMEM((N + 32,), jnp.int32),
        pltpu.VMEM((S1 + 16,), jnp.int32),
        pltpu.VMEM((N + 32,), jnp.int32),
        pltpu.VMEM((PC * S1,), jnp.int32),
        pltpu.VMEM((PC * S2,), jnp.int32),
    ],
)
def _ballquery_sc(xs_hbm, ys_hbm, zs_hbm, nxs_hbm, nys_hbm, nzs_hbm,
                  out1_hbm, out2_hbm, xv, yv, zv, nxv, nyv, nzv,
                  ca1, ca2, cb1, cb2, o1, o2):
    wid = lax.axis_index("s") * NC + lax.axis_index("c")
    tiles_per_b = NW // B
    b = wid // tiles_per_b
    p0 = (wid % tiles_per_b) * PC
    pltpu.sync_copy(xs_hbm.at[b], xv)
    pltpu.sync_copy(ys_hbm.at[b], yv)
    pltpu.sync_copy(zs_hbm.at[b], zv)
    pltpu.sync_copy(nxs_hbm.at[b, pl.ds(p0, PC)], nxv)
    pltpu.sync_copy(nys_hbm.at[b, pl.ds(p0, PC)], nyv)
    pltpu.sync_copy(nzs_hbm.at[b, pl.ds(p0, PC)], nzv)
    lane = lax.iota(jnp.int32, 16)
    gbase = b * N

    HP = PC // 2

    def per_pair(i, _):
        ia, ib = i, i + HP
        cxa = plsc.load_gather(nxv, [jnp.full((16,), ia, jnp.int32)])
        cya = plsc.load_gather(nyv, [jnp.full((16,), ia, jnp.int32)])
        cza = plsc.load_gather(nzv, [jnp.full((16,), ia, jnp.int32)])
        cxb = plsc.load_gather(nxv, [jnp.full((16,), ib, jnp.int32)])
        cyb = plsc.load_gather(nyv, [jnp.full((16,), ib, jnp.int32)])
        czb = plsc.load_gather(nzv, [jnp.full((16,), ib, jnp.int32)])

        def cond(state):
            c, a1, a2, b1, b2 = state
            return (c < N // 32) & ((a1 < S1) | (a2 < S2)
                                    | (b1 < S1) | (b2 < S2))

        def body(state):
            c, a1, a2, b1, b2 = state
            for u in range(2):
                off = c * 32 + u * 16
                xc = xv[pl.ds(off, 16)]
                yc = yv[pl.ds(off, 16)]
                zc = zv[pl.ds(off, 16)]
                jv = off + lane
                dxa = cxa - xc
                dya = cya - yc
                dza = cza - zc
                d2a = (dxa * dxa + dya * dya) + dza * dza
                dxb = cxb - xc
                dyb = cyb - yc
                dzb = czb - zc
                d2b = (dxb * dxb + dyb * dyb) + dzb * dzb
                ma1 = d2a < R1SQ
                ma2 = d2a < R2SQ
                mb1 = d2b < R1SQ
                mb2 = d2b < R2SQ
                plsc.store_compressed(ca2.at[pl.ds(a2, 16)], jv, mask=ma2)
                plsc.store_compressed(cb2.at[pl.ds(b2, 16)], jv, mask=mb2)
                a1 = a1 + plsc.all_reduce_population_count(ma1)[0]
                a2 = a2 + plsc.all_reduce_population_count(ma2)[0]
                b1 = b1 + plsc.all_reduce_population_count(mb1)[0]
                b2 = b2 + plsc.all_reduce_population_count(mb2)[0]
            return (c + 1, a1, a2, b1, b2)

        _, a1, a2, b1, b2 = lax.while_loop(cond, body, (0, 0, 0, 0, 0))

        true16 = lane < 16
        for (cand1, cand2, cxc, cyc, czc, n2, ci) in (
                (ca1, ca2, cxa, cya, cza, a2, ia),
                (cb1, cb2, cxb, cyb, czb, b2, ib)):
            f2 = jnp.full((16,), cand2[pl.ds(0, 16)][0], jnp.int32)
            plsc.store_compressed(cand2.at[pl.ds(n2, 16)], f2, mask=true16)
            plsc.store_compressed(cand2.at[pl.ds(n2 + 16, 16)], f2,
                                  mask=true16)

            def pcond(s, _n2=n2):
                k, n1 = s
                return (k * 16 < _n2) & (n1 < S1)

            def pbody(s, _c1=cand1, _c2=cand2, _cx=cxc, _cy=cyc, _cz=czc):
                k, n1 = s
                jc = _c2[pl.ds(k * 16, 16)]
                dx = _cx - plsc.load_gather(xv, [jc])
                dy = _cy - plsc.load_gather(yv, [jc])
                dz = _cz - plsc.load_gather(zv, [jc])
                d2 = (dx * dx + dy * dy) + dz * dz
                m1 = d2 < R1SQ
                plsc.store_compressed(_c1.at[pl.ds(n1, 16)], jc, mask=m1)
                return (k + 1, n1 + plsc.all_reduce_population_count(m1)[0])

            _, n1 = lax.while_loop(pcond, pbody, (0, 0))

            f1 = jnp.full((16,), cand1[pl.ds(0, 16)][0], jnp.int32)
            plsc.store_compressed(cand1.at[pl.ds(jnp.minimum(n1, S1), 16)],
                                  f1, mask=true16)
            o1[pl.ds(ci * S1, 16)] = cand1[pl.ds(0, 16)] + gbase
            for k in range(S2 // 16):
                o2[pl.ds(ci * S2 + k * 16, 16)] = (cand2[pl.ds(k * 16, 16)]
                                                   + gbase)
        return 0

    lax.fori_loop(0, HP, per_pair, 0)
    pltpu.sync_copy(o1, out1_hbm.at[pl.ds(wid * PC * S1, PC * S1)])
    pltpu.sync_copy(o2, out2_hbm.at[pl.ds(wid * PC * S2, PC * S2)])


TT = T1 + T2
GCHUNK = 128


@functools.partial(
    pl.kernel,
    out_type=jax.ShapeDtypeStruct((TT, D), jnp.float32),
    mesh=_SC_MESH,
    compiler_params=pltpu.CompilerParams(
        needs_layout_passes=False, use_tc_tiling_on_sc=False),
    scratch_types=[
        pltpu.VMEM((GCHUNK,), jnp.int32),
        pltpu.VMEM((GCHUNK,), jnp.int32),
        pltpu.VMEM((GCHUNK, D), jnp.float32),
        pltpu.VMEM((GCHUNK, D), jnp.float32),
        pltpu.SemaphoreType.DMA,
        pltpu.SemaphoreType.DMA,
    ],
)
def _gather_sc(table_hbm, idx_hbm, out_hbm, idx_v0, idx_v1, rows_v0, rows_v1,
               sem0, sem1):
    wid = lax.axis_index("s") * NC + lax.axis_index("c")
    per_w = TT // NW
    nchunks = per_w // GCHUNK
    base = wid * per_w
    idx_v = (idx_v0, idx_v1)
    rows_v = (rows_v0, rows_v1)
    sems = (sem0, sem1)

    for b in range(2):
        pltpu.sync_copy(idx_hbm.at[pl.ds(base + b * GCHUNK, GCHUNK)],
                        idx_v[b])
        pltpu.async_copy(table_hbm.at[idx_v[b]], rows_v[b], sems[b])

    def outer(g, _):
        for b in range(2):
            k = g * 2 + b
            pltpu.make_async_copy(table_hbm.at[idx_v[b]], rows_v[b],
                                  sems[b]).wait()
            pltpu.sync_copy(rows_v[b], out_hbm.at[pl.ds(base + k * GCHUNK,
                                                        GCHUNK)])

            @pl.when(k + 2 < nchunks)
            def _():
                pltpu.sync_copy(
                    idx_hbm.at[pl.ds(base + (k + 2) * GCHUNK, GCHUNK)],
                    idx_v[b])
                pltpu.async_copy(table_hbm.at[idx_v[b]], rows_v[b], sems[b])

        return 0

    lax.fori_loop(0, nchunks // 2, outer, 0)


def _bf(v):
    return v.astype(jnp.bfloat16).astype(jnp.float32)


def _h1_from_tile(gat, aux, w1p):
    raw = gat[:, 0:3]
    delta = raw - aux[:, 0:3]
    d0 = delta[:, 0:1]
    d1 = delta[:, 1:2]
    d2c = delta[:, 2:3]
    dist = jnp.sqrt((d0 * d0 + d1 * d1) + d2c * d2c + 1e-12)
    h1 = w1p[15:16, :]
    h1 = h1 + _bf(dist) * _bf(w1p[0:1, :])
    for k in range(3):
        h1 = h1 + _bf(aux[:, 3 + k:4 + k]) * _bf(w1p[1 + k:2 + k, :])
    for k in range(3):
        h1 = h1 + _bf(gat[:, k:k + 1]) * _bf(w1p[4 + k:5 + k, :])
    h1 = h1 + _bf(d0) * _bf(w1p[7:8, :])
    h1 = h1 + _bf(d1) * _bf(w1p[8:9, :])
    h1 = h1 + _bf(d2c) * _bf(w1p[9:10, :])
    return h1, delta


def _d1_body(gat_ref, aux_ref, w1p_ref, s_ref):
    h1, _ = _h1_from_tile(gat_ref[:, :], aux_ref[:, :], w1p_ref[:, :])
    p0 = jnp.sum(h1, axis=0, keepdims=True)
    p1 = jnp.sum(h1 * h1, axis=0, keepdims=True)
    part = jnp.concatenate([p0, p1], axis=0)

    @pl.when(pl.program_id(0) == 0)
    def _():
        s_ref[:, :] = part

    @pl.when(pl.program_id(0) != 0)
    def _():
        s_ref[:, :] = s_ref[:, :] + part


def _d2_body(gat_ref, aux_ref, w1p_ref, st_ref, w2p_ref, b2p_ref,
             ymax_ref, ys_ref, *, rows, s, t_count):
    h1, delta = _h1_from_tile(gat_ref[:, :], aux_ref[:, :], w1p_ref[:, :])
    st = st_ref[:, :]
    mu = st[0:1, :] * (1.0 / t_count)
    var = st[1:2, :] * (1.0 / t_count) - mu * mu
    rs = 1.0 / jnp.sqrt(var + EPS)
    h1n = jnp.maximum((h1 - mu) * rs, 0.0)
    h2 = jnp.dot(_bf(h1n), _bf(w2p_ref[:, :]),
                 preferred_element_type=jnp.float32,
                 precision=lax.Precision.HIGHEST)
    h2 = h2 + b2p_ref[:, :]
    x = jnp.concatenate(
        [delta, gat_ref[:, 16:80], jnp.zeros((rows, 128 - CIN), jnp.float32)],
        axis=1)
    y = h2 * x
    p0 = jnp.sum(y, axis=0, keepdims=True)
    p1 = jnp.sum(y * y, axis=0, keepdims=True)
    part = jnp.concatenate([p0, p1], axis=0)

    @pl.when(pl.program_id(0) == 0)
    def _():
        ys_ref[:, :] = part

    @pl.when(pl.program_id(0) != 0)
    def _():
        ys_ref[:, :] = ys_ref[:, :] + part

    for k in range(rows // s):
        ymax_ref[k:k + 1, :] = jnp.max(y[k * s:(k + 1) * s, :], axis=0,
                                       keepdims=True)


def _e_body(ymax_ref, ys_ref, wcrp_ref, bcr_ref, out_ref, *, t_count):
    st = ys_ref[:, :]
    mu = st[0:1, :] * (1.0 / t_count)
    var = st[1:2, :] * (1.0 / t_count) - mu * mu
    rs = 1.0 / jnp.sqrt(var + EPS)
    yn = jnp.maximum((ymax_ref[:, :] - mu) * rs, 0.0)
    z = jnp.dot(_bf(yn), _bf(wcrp_ref[:, :]),
                preferred_element_type=jnp.float32,
                precision=lax.Precision.HIGHEST)
    z = z + bcr_ref[:, :]
    muz = jnp.sum(z, axis=0, keepdims=True) * (1.0 / (B * P))
    varz = jnp.sum(z * z, axis=0, keepdims=True) * (1.0 / (B * P)) - muz * muz
    out_ref[:, :] = jnp.maximum((z - muz) / jnp.sqrt(varz + EPS), 0.0)


def _rsconv_scale(gat, aux, w1p, w2p, b2p, wcrp, bcrp, s):
    t_rows = gat.shape[0]
    rows = 512
    grid = (t_rows // rows,)
    stats = pl.pallas_call(
        _d1_body,
        grid=grid,
        in_specs=[
            pl.BlockSpec((rows, D), lambda i: (i, 0)),
            pl.BlockSpec((rows, 8), lambda i: (i, 0)),
            pl.BlockSpec((16, CMID), lambda i: (0, 0)),
        ],
        out_specs=pl.BlockSpec((2, CMID), lambda i: (0, 0)),
        out_shape=jax.ShapeDtypeStruct((2, CMID), jnp.float32),
    )(gat, aux, w1p)
    ymax, ys = pl.pallas_call(
        functools.partial(_d2_body, rows=rows, s=s, t_count=float(t_rows)),
        grid=grid,
        in_specs=[
            pl.BlockSpec((rows, D), lambda i: (i, 0)),
            pl.BlockSpec((rows, 8), lambda i: (i, 0)),
            pl.BlockSpec((16, CMID), lambda i: (0, 0)),
            pl.BlockSpec((2, CMID), lambda i: (0, 0)),
            pl.BlockSpec((CMID, 128), lambda i: (0, 0)),
            pl.BlockSpec((1, 128), lambda i: (0, 0)),
        ],
        out_specs=[
            pl.BlockSpec((rows // s, 128), lambda i: (i, 0)),
            pl.BlockSpec((2, 128), lambda i: (0, 0)),
        ],
        out_shape=[
            jax.ShapeDtypeStruct((B * P, 128), jnp.float32),
            jax.ShapeDtypeStruct((2, 128), jnp.float32),
        ],
    )(gat, aux, w1p, stats, w2p, b2p)
    out = pl.pallas_call(
        functools.partial(_e_body, t_count=float(t_rows)),
        in_specs=[
            pl.BlockSpec((B * P, 128), lambda: (0, 0)),
            pl.BlockSpec((2, 128), lambda: (0, 0)),
            pl.BlockSpec((128, 128), lambda: (0, 0)),
            pl.BlockSpec((1, 128), lambda: (0, 0)),
        ],
        out_specs=pl.BlockSpec((B * P, 128), lambda: (0, 0)),
        out_shape=jax.ShapeDtypeStruct((B * P, 128), jnp.float32),
    )(ymax, ys, wcrp, bcrp)
    return out


def kernel(xyz, features, w1, b1, w2, b2, wcr, bcr):
    xs = xyz[:, :, 0]
    ys = xyz[:, :, 1]
    zs = xyz[:, :, 2]
    nx, ny, nz = _fps(xs, ys, zs)
    new_xyz = jnp.stack([nx, ny, nz], axis=-1)

    idx1, idx2 = _ballquery_sc(xs, ys, zs, nx, ny, nz)

    table = jnp.concatenate(
        [xyz, jnp.zeros((B, N, 13), jnp.float32),
         jnp.transpose(features, (0, 2, 1))], axis=2).reshape(B * N, D)
    gat = _gather_sc(table, jnp.concatenate([idx1, idx2]))
    gat1 = gat[:T1]
    gat2 = gat[T1:]

    newx_flat = new_xyz.reshape(B * P, 3)

    def make_aux(g, s):
        nrep = jnp.repeat(newx_flat, s, axis=0)
        xi = jnp.repeat(g.reshape(B * P, s, D)[:, 0, 0:3], s, axis=0)
        return jnp.concatenate(
            [nrep, xi, jnp.zeros((B * P * s, 2), jnp.float32)], axis=1)

    w1p = jnp.zeros((16, CMID), jnp.float32).at[0:10].set(w1.T).at[15].set(b1)
    w2p = jnp.zeros((CMID, 128), jnp.float32).at[:, 0:CIN].set(w2.T)
    b2p = jnp.zeros((1, 128), jnp.float32).at[:, 0:CIN].set(b2)
    wcrp = jnp.zeros((128, 128), jnp.float32).at[0:CIN].set(wcr.T)
    bcrp = bcr.reshape(1, 128)

    o1 = _rsconv_scale(gat1, make_aux(gat1, S1), w1p, w2p, b2p, wcrp, bcrp, S1)
    o2 = _rsconv_scale(gat2, make_aux(gat2, S2), w1p, w2p, b2p, wcrp, bcrp, S2)
    out = jnp.concatenate([
        o1.reshape(B, P, COUT).transpose(0, 2, 1),
        o2.reshape(B, P, COUT).transpose(0, 2, 1),
    ], axis=1)
    return (new_xyz, out)
